# Initial kernel scaffold; baseline (speedup 1.0000x reference)
import jax
import jax.numpy as jnp
from jax.experimental import pallas as pl


def kernel(x, edge_index, Wl1, bl1, Wr1, Wl2, bl2, Wr2, Wl3, bl3, Wr3, W4, b4):
    raise NotImplementedError("write your pallas kernel here")


# speedup vs baseline: 8.8708x; 8.8708x over previous
import functools

import jax
import jax.numpy as jnp
from jax import lax
from jax.experimental import pallas as pl
from jax.experimental.pallas import tpu as pltpu
from jax.experimental.pallas import tpu_sc as plsc

_N = 10000
_E = 320000
_NC = 2
_NS = 16
_NW = _NC * _NS
_CK = 80
_NCHT = _E // _CK
_NCH = _NCHT // _NW
_NCHP = 128
_RPT0 = 632
_RPT1 = _N - 15 * _RPT0
_BM = 1000


def _make_segsum(W):
    mesh = plsc.VectorSubcoreMesh(core_axis_name="c", subcore_axis_name="s")

    @functools.partial(
        pl.kernel,
        mesh=mesh,
        compiler_params=pltpu.CompilerParams(use_tc_tiling_on_sc=False),
        out_type=jax.ShapeDtypeStruct((_NC, _N, W), jnp.float32),
        scratch_types=[
            pltpu.VMEM((_NCHP, _CK), jnp.int32),
            pltpu.VMEM((_NCHP, _CK), jnp.int32),
            pltpu.VMEM((_CK, W), jnp.float32),
            pltpu.VMEM_SHARED((_N, W), jnp.float32),
            pltpu.SemaphoreType.DMA,
        ],
    )
    def seg(table, src2d, dst2d, zeros, out, sidx, didx, rows, acc, sem):
        c = lax.axis_index("c")
        s = lax.axis_index("s")
        wid = c * _NS + s
        row0 = s * _RPT0
        @pl.when(s < 15)
        def _():
            pltpu.sync_copy(zeros.at[pl.ds(0, _RPT0)],
                            acc.at[pl.ds(row0, _RPT0)])

        @pl.when(s == 15)
        def _():
            pltpu.sync_copy(zeros.at[pl.ds(0, _RPT1)],
                            acc.at[pl.ds(row0, _RPT1)])

        pltpu.sync_copy(src2d.at[pl.ds(wid * _NCHP, _NCHP)], sidx)
        pltpu.sync_copy(dst2d.at[pl.ds(wid * _NCHP, _NCHP)], didx)
        plsc.subcore_barrier()

        def body(i, carry):
            pltpu.async_copy(table.at[sidx.at[i]], rows, sem).wait()
            pltpu.sync_copy(rows, acc.at[didx.at[i]], add=True)
            return carry

        lax.fori_loop(0, _NCH, body, 0)
        plsc.subcore_barrier()

        @pl.when(s < 15)
        def _():
            pltpu.sync_copy(acc.at[pl.ds(row0, _RPT0)],
                            out.at[c, pl.ds(row0, _RPT0)])

        @pl.when(s == 15)
        def _():
            pltpu.sync_copy(acc.at[pl.ds(row0, _RPT1)],
                            out.at[c, pl.ds(row0, _RPT1)])

    return seg


_segsum80 = _make_segsum(80)
_segsum64 = _make_segsum(64)


def _tc_a(x, wl1t, wr1t):
    def body(x_ref, wl_ref, wr_ref, y_ref, r_ref):
        xb = x_ref[...]
        y = jnp.dot(xb, wl_ref[...], preferred_element_type=jnp.float32)
        extra = (lax.broadcasted_iota(jnp.int32, (_BM, 16), 1) == 0)
        y_ref[...] = jnp.concatenate([y, extra.astype(jnp.float32)], axis=1)
        r_ref[...] = jnp.dot(xb, wr_ref[...], preferred_element_type=jnp.float32)

    return pl.pallas_call(
        body,
        grid=(_N // _BM,),
        in_specs=[
            pl.BlockSpec((_BM, 128), lambda i: (i, 0)),
            pl.BlockSpec((128, 64), lambda i: (0, 0)),
            pl.BlockSpec((128, 64), lambda i: (0, 0)),
        ],
        out_specs=[
            pl.BlockSpec((_BM, 80), lambda i: (i, 0)),
            pl.BlockSpec((_BM, 64), lambda i: (i, 0)),
        ],
        out_shape=[
            jax.ShapeDtypeStruct((_N, 80), jnp.float32),
            jax.ShapeDtypeStruct((_N, 64), jnp.float32),
        ],
    )(x, wl1t, wr1t)


def _tc_b(p1, bl1, r1):
    def body(p_ref, bl_ref, r_ref, h_ref, inv_ref):
        p = p_ref[0] + p_ref[1]
        lane = lax.broadcasted_iota(jnp.int32, (_BM, 80), 1)
        cnt = jnp.sum(jnp.where(lane == 64, p, 0.0), axis=1, keepdims=True)
        inv = 1.0 / jnp.maximum(cnt, 1.0)
        h = p[:, :64] * inv + bl_ref[...] + r_ref[...]
        h_ref[...] = jnp.maximum(h, 0.0)
        inv_ref[...] = inv

    return pl.pallas_call(
        body,
        grid=(_N // _BM,),
        in_specs=[
            pl.BlockSpec((_NC, _BM, 80), lambda i: (0, i, 0)),
            pl.BlockSpec((1, 64), lambda i: (0, 0)),
            pl.BlockSpec((_BM, 64), lambda i: (i, 0)),
        ],
        out_specs=[
            pl.BlockSpec((_BM, 64), lambda i: (i, 0)),
            pl.BlockSpec((_BM, 1), lambda i: (i, 0)),
        ],
        out_shape=[
            jax.ShapeDtypeStruct((_N, 64), jnp.float32),
            jax.ShapeDtypeStruct((_N, 1), jnp.float32),
        ],
    )(p1, bl1, r1)


def _tc_c(p2, inv, h1, wl2t, bl2, wr2t, wl3t, wr3t):
    def body(p_ref, inv_ref, h1_ref, wl2_ref, bl2_ref, wr2_ref, wl3_ref,
             wr3_ref, y3_ref, r3_ref):
        m2 = (p_ref[0] + p_ref[1]) * inv_ref[...]
        h2 = jnp.dot(m2, wl2_ref[...], preferred_element_type=jnp.float32)
        h2 = h2 + bl2_ref[...]
        h2 = h2 + jnp.dot(h1_ref[...], wr2_ref[...],
                          preferred_element_type=jnp.float32)
        h2 = jnp.maximum(h2, 0.0)
        y3_ref[...] = jnp.dot(h2, wl3_ref[...], preferred_element_type=jnp.float32)
        r3_ref[...] = jnp.dot(h2, wr3_ref[...], preferred_element_type=jnp.float32)

    return pl.pallas_call(
        body,
        grid=(_N // _BM,),
        in_specs=[
            pl.BlockSpec((_NC, _BM, 64), lambda i: (0, i, 0)),
            pl.BlockSpec((_BM, 1), lambda i: (i, 0)),
            pl.BlockSpec((_BM, 64), lambda i: (i, 0)),
            pl.BlockSpec((64, 128), lambda i: (0, 0)),
            pl.BlockSpec((1, 128), lambda i: (0, 0)),
            pl.BlockSpec((64, 128), lambda i: (0, 0)),
            pl.BlockSpec((128, 64), lambda i: (0, 0)),
            pl.BlockSpec((128, 64), lambda i: (0, 0)),
        ],
        out_specs=[
            pl.BlockSpec((_BM, 64), lambda i: (i, 0)),
            pl.BlockSpec((_BM, 64), lambda i: (i, 0)),
        ],
        out_shape=[
            jax.ShapeDtypeStruct((_N, 64), jnp.float32),
            jax.ShapeDtypeStruct((_N, 64), jnp.float32),
        ],
    )(p2, inv, h1, wl2t, bl2, wr2t, wl3t, wr3t)


def _tc_d(p3, inv, r3, bl3, w4t, b4):
    def body(p_ref, inv_ref, r_ref, bl_ref, w4_ref, b4_ref, o_ref):
        h3 = (p_ref[0] + p_ref[1]) * inv_ref[...] + bl_ref[...] + r_ref[...]
        h3 = jnp.maximum(h3, 0.0)
        o_ref[...] = jnp.dot(h3, w4_ref[...],
                             preferred_element_type=jnp.float32) + b4_ref[...]

    return pl.pallas_call(
        body,
        grid=(_N // _BM,),
        in_specs=[
            pl.BlockSpec((_NC, _BM, 64), lambda i: (0, i, 0)),
            pl.BlockSpec((_BM, 1), lambda i: (i, 0)),
            pl.BlockSpec((_BM, 64), lambda i: (i, 0)),
            pl.BlockSpec((1, 64), lambda i: (0, 0)),
            pl.BlockSpec((64, 128), lambda i: (0, 0)),
            pl.BlockSpec((1, 128), lambda i: (0, 0)),
        ],
        out_specs=pl.BlockSpec((_BM, 128), lambda i: (i, 0)),
        out_shape=jax.ShapeDtypeStruct((_N, 128), jnp.float32),
    )(p3, inv, r3, bl3, w4t, b4)


def kernel(x, edge_index, Wl1, bl1, Wr1, Wl2, bl2, Wr2, Wl3, bl3, Wr3, W4, b4):
    e4 = edge_index.reshape(2, _NW, _NCH, _CK)
    epad = jnp.concatenate(
        [e4, jnp.zeros((2, _NW, _NCHP - _NCH, _CK), jnp.int32)], axis=2
    ).reshape(2, _NW * _NCHP, _CK)
    src = epad[0]
    dst = epad[1]
    zeros80 = jnp.zeros((_RPT0, 80), jnp.float32)
    zeros64 = jnp.zeros((_RPT0, 64), jnp.float32)
    w4t = jnp.zeros((64, 128), jnp.float32).at[:, :3].set(W4.T)
    b4p = jnp.zeros((1, 128), jnp.float32).at[0, :3].set(b4)

    y1, r1 = _tc_a(x, Wl1.T, Wr1.T)
    p1 = _segsum80(y1, src, dst, zeros80)
    h1, inv = _tc_b(p1, bl1.reshape(1, 64), r1)
    p2 = _segsum64(h1, src, dst, zeros64)
    y3, r3 = _tc_c(p2, inv, h1, Wl2.T, bl2.reshape(1, 128), Wr2.T,
                   Wl3.T, Wr3.T)
    p3 = _segsum64(y3, src, dst, zeros64)
    out = _tc_d(p3, inv, r3, bl3.reshape(1, 64), w4t, b4p)
    return out[:, :3]


# speedup vs baseline: 13.3231x; 1.5019x over previous
import functools

import jax
import jax.numpy as jnp
from jax import lax
from jax.experimental import pallas as pl
from jax.experimental.pallas import tpu as pltpu
from jax.experimental.pallas import tpu_sc as plsc

_N = 10000
_E = 320000
_NC = 2
_NS = 16
_NW = _NC * _NS
_CK = 80
_NCHT = _E // _CK
_NCH = _NCHT // _NW
_NCHP = 128
_RPT0 = 632
_RPT1 = _N - 15 * _RPT0
_BM = 1000


def _make_segsum(W):
    mesh = plsc.VectorSubcoreMesh(core_axis_name="c", subcore_axis_name="s")

    @functools.partial(
        pl.kernel,
        mesh=mesh,
        compiler_params=pltpu.CompilerParams(use_tc_tiling_on_sc=False),
        out_type=jax.ShapeDtypeStruct((_NC, _N, W), jnp.float32),
        scratch_types=[
            pltpu.VMEM((_NCHP, _CK), jnp.int32),
            pltpu.VMEM((_NCHP, _CK), jnp.int32),
            pltpu.VMEM((_CK, W), jnp.float32),
            pltpu.VMEM((_CK, W), jnp.float32),
            pltpu.VMEM_SHARED((_N, W), jnp.float32),
            pltpu.SemaphoreType.DMA,
            pltpu.SemaphoreType.DMA,
        ],
    )
    def seg(table, src2d, dst2d, zeros, out, sidx, didx, rows0, rows1, acc,
            sem0, sem1):
        c = lax.axis_index("c")
        s = lax.axis_index("s")
        wid = c * _NS + s
        row0 = s * _RPT0
        @pl.when(s < 15)
        def _():
            pltpu.sync_copy(zeros.at[pl.ds(0, _RPT0)],
                            acc.at[pl.ds(row0, _RPT0)])

        @pl.when(s == 15)
        def _():
            pltpu.sync_copy(zeros.at[pl.ds(0, _RPT1)],
                            acc.at[pl.ds(row0, _RPT1)])

        pltpu.sync_copy(src2d.at[pl.ds(wid * _NCHP, _NCHP)], sidx)
        pltpu.sync_copy(dst2d.at[pl.ds(wid * _NCHP, _NCHP)], didx)
        plsc.subcore_barrier()

        pltpu.async_copy(table.at[sidx.at[0]], rows0, sem0)

        def body(i, carry):
            even = (i % 2) == 0
            nxt = i + 1

            @pl.when(jnp.logical_and(nxt < _NCH, even))
            def _():
                pltpu.async_copy(table.at[sidx.at[nxt]], rows1, sem1)

            @pl.when(jnp.logical_and(nxt < _NCH, jnp.logical_not(even)))
            def _():
                pltpu.async_copy(table.at[sidx.at[nxt]], rows0, sem0)

            @pl.when(even)
            def _():
                pltpu.make_async_copy(table.at[sidx.at[i]], rows0, sem0).wait()
                pltpu.sync_copy(rows0, acc.at[didx.at[i]], add=True)

            @pl.when(jnp.logical_not(even))
            def _():
                pltpu.make_async_copy(table.at[sidx.at[i]], rows1, sem1).wait()
                pltpu.sync_copy(rows1, acc.at[didx.at[i]], add=True)

            return carry

        lax.fori_loop(0, _NCH, body, 0)
        plsc.subcore_barrier()

        @pl.when(s < 15)
        def _():
            pltpu.sync_copy(acc.at[pl.ds(row0, _RPT0)],
                            out.at[c, pl.ds(row0, _RPT0)])

        @pl.when(s == 15)
        def _():
            pltpu.sync_copy(acc.at[pl.ds(row0, _RPT1)],
                            out.at[c, pl.ds(row0, _RPT1)])

    return seg


_segsum80 = _make_segsum(80)
_segsum64 = _make_segsum(64)


def _tc_a(x, wl1t, wr1t):
    def body(x_ref, wl_ref, wr_ref, y_ref, r_ref):
        xb = x_ref[...]
        y = jnp.dot(xb, wl_ref[...], preferred_element_type=jnp.float32)
        extra = (lax.broadcasted_iota(jnp.int32, (_BM, 16), 1) == 0)
        y_ref[...] = jnp.concatenate([y, extra.astype(jnp.float32)], axis=1)
        r_ref[...] = jnp.dot(xb, wr_ref[...], preferred_element_type=jnp.float32)

    return pl.pallas_call(
        body,
        grid=(_N // _BM,),
        in_specs=[
            pl.BlockSpec((_BM, 128), lambda i: (i, 0)),
            pl.BlockSpec((128, 64), lambda i: (0, 0)),
            pl.BlockSpec((128, 64), lambda i: (0, 0)),
        ],
        out_specs=[
            pl.BlockSpec((_BM, 80), lambda i: (i, 0)),
            pl.BlockSpec((_BM, 64), lambda i: (i, 0)),
        ],
        out_shape=[
            jax.ShapeDtypeStruct((_N, 80), jnp.float32),
            jax.ShapeDtypeStruct((_N, 64), jnp.float32),
        ],
    )(x, wl1t, wr1t)


def _tc_b(p1, bl1, r1):
    def body(p_ref, bl_ref, r_ref, h_ref, inv_ref):
        p = p_ref[0] + p_ref[1]
        lane = lax.broadcasted_iota(jnp.int32, (_BM, 80), 1)
        cnt = jnp.sum(jnp.where(lane == 64, p, 0.0), axis=1, keepdims=True)
        inv = 1.0 / jnp.maximum(cnt, 1.0)
        h = p[:, :64] * inv + bl_ref[...] + r_ref[...]
        h_ref[...] = jnp.maximum(h, 0.0)
        inv_ref[...] = inv

    return pl.pallas_call(
        body,
        grid=(_N // _BM,),
        in_specs=[
            pl.BlockSpec((_NC, _BM, 80), lambda i: (0, i, 0)),
            pl.BlockSpec((1, 64), lambda i: (0, 0)),
            pl.BlockSpec((_BM, 64), lambda i: (i, 0)),
        ],
        out_specs=[
            pl.BlockSpec((_BM, 64), lambda i: (i, 0)),
            pl.BlockSpec((_BM, 1), lambda i: (i, 0)),
        ],
        out_shape=[
            jax.ShapeDtypeStruct((_N, 64), jnp.float32),
            jax.ShapeDtypeStruct((_N, 1), jnp.float32),
        ],
    )(p1, bl1, r1)


def _tc_c(p2, inv, h1, wl2t, bl2, wr2t, wl3t, wr3t):
    def body(p_ref, inv_ref, h1_ref, wl2_ref, bl2_ref, wr2_ref, wl3_ref,
             wr3_ref, y3_ref, r3_ref):
        m2 = (p_ref[0] + p_ref[1]) * inv_ref[...]
        h2 = jnp.dot(m2, wl2_ref[...], preferred_element_type=jnp.float32)
        h2 = h2 + bl2_ref[...]
        h2 = h2 + jnp.dot(h1_ref[...], wr2_ref[...],
                          preferred_element_type=jnp.float32)
        h2 = jnp.maximum(h2, 0.0)
        y3_ref[...] = jnp.dot(h2, wl3_ref[...], preferred_element_type=jnp.float32)
        r3_ref[...] = jnp.dot(h2, wr3_ref[...], preferred_element_type=jnp.float32)

    return pl.pallas_call(
        body,
        grid=(_N // _BM,),
        in_specs=[
            pl.BlockSpec((_NC, _BM, 64), lambda i: (0, i, 0)),
            pl.BlockSpec((_BM, 1), lambda i: (i, 0)),
            pl.BlockSpec((_BM, 64), lambda i: (i, 0)),
            pl.BlockSpec((64, 128), lambda i: (0, 0)),
            pl.BlockSpec((1, 128), lambda i: (0, 0)),
            pl.BlockSpec((64, 128), lambda i: (0, 0)),
            pl.BlockSpec((128, 64), lambda i: (0, 0)),
            pl.BlockSpec((128, 64), lambda i: (0, 0)),
        ],
        out_specs=[
            pl.BlockSpec((_BM, 64), lambda i: (i, 0)),
            pl.BlockSpec((_BM, 64), lambda i: (i, 0)),
        ],
        out_shape=[
            jax.ShapeDtypeStruct((_N, 64), jnp.float32),
            jax.ShapeDtypeStruct((_N, 64), jnp.float32),
        ],
    )(p2, inv, h1, wl2t, bl2, wr2t, wl3t, wr3t)


def _tc_d(p3, inv, r3, bl3, w4t, b4):
    def body(p_ref, inv_ref, r_ref, bl_ref, w4_ref, b4_ref, o_ref):
        h3 = (p_ref[0] + p_ref[1]) * inv_ref[...] + bl_ref[...] + r_ref[...]
        h3 = jnp.maximum(h3, 0.0)
        o_ref[...] = jnp.dot(h3, w4_ref[...],
                             preferred_element_type=jnp.float32) + b4_ref[...]

    return pl.pallas_call(
        body,
        grid=(_N // _BM,),
        in_specs=[
            pl.BlockSpec((_NC, _BM, 64), lambda i: (0, i, 0)),
            pl.BlockSpec((_BM, 1), lambda i: (i, 0)),
            pl.BlockSpec((_BM, 64), lambda i: (i, 0)),
            pl.BlockSpec((1, 64), lambda i: (0, 0)),
            pl.BlockSpec((64, 128), lambda i: (0, 0)),
            pl.BlockSpec((1, 128), lambda i: (0, 0)),
        ],
        out_specs=pl.BlockSpec((_BM, 128), lambda i: (i, 0)),
        out_shape=jax.ShapeDtypeStruct((_N, 128), jnp.float32),
    )(p3, inv, r3, bl3, w4t, b4)


def kernel(x, edge_index, Wl1, bl1, Wr1, Wl2, bl2, Wr2, Wl3, bl3, Wr3, W4, b4):
    e4 = edge_index.reshape(2, _NW, _NCH, _CK)
    epad = jnp.concatenate(
        [e4, jnp.zeros((2, _NW, _NCHP - _NCH, _CK), jnp.int32)], axis=2
    ).reshape(2, _NW * _NCHP, _CK)
    src = epad[0]
    dst = epad[1]
    zeros80 = jnp.zeros((_RPT0, 80), jnp.float32)
    zeros64 = jnp.zeros((_RPT0, 64), jnp.float32)
    w4t = jnp.zeros((64, 128), jnp.float32).at[:, :3].set(W4.T)
    b4p = jnp.zeros((1, 128), jnp.float32).at[0, :3].set(b4)

    y1, r1 = _tc_a(x, Wl1.T, Wr1.T)
    p1 = _segsum80(y1, src, dst, zeros80)
    h1, inv = _tc_b(p1, bl1.reshape(1, 64), r1)
    p2 = _segsum64(h1, src, dst, zeros64)
    y3, r3 = _tc_c(p2, inv, h1, Wl2.T, bl2.reshape(1, 128), Wr2.T,
                   Wl3.T, Wr3.T)
    p3 = _segsum64(y3, src, dst, zeros64)
    out = _tc_d(p3, inv, r3, bl3.reshape(1, 64), w4t, b4p)
    return out[:, :3]


# speedup vs baseline: 14.2949x; 1.0729x over previous
import functools

import jax
import jax.numpy as jnp
from jax import lax
from jax.experimental import pallas as pl
from jax.experimental.pallas import tpu as pltpu
from jax.experimental.pallas import tpu_sc as plsc

_N = 10000
_E = 320000
_NC = 2
_NS = 16
_NW = _NC * _NS
_CK = 80
_NCHT = _E // _CK
_NCH = _NCHT // _NW
_NCHP = 128
_RPT0 = 632
_RPT1 = _N - 15 * _RPT0
_BM = 1000


def _make_segsum(W):
    mesh = plsc.VectorSubcoreMesh(core_axis_name="c", subcore_axis_name="s")

    @functools.partial(
        pl.kernel,
        mesh=mesh,
        compiler_params=pltpu.CompilerParams(use_tc_tiling_on_sc=False),
        out_type=jax.ShapeDtypeStruct((_NC, _N, W), jnp.float32),
        scratch_types=[
            pltpu.VMEM((_NCHP, _CK), jnp.int32),
            pltpu.VMEM((_NCHP, _CK), jnp.int32),
            pltpu.VMEM((_CK, W), jnp.float32),
            pltpu.VMEM((_CK, W), jnp.float32),
            pltpu.VMEM((_CK, W), jnp.float32),
            pltpu.VMEM((_CK, W), jnp.float32),
            pltpu.VMEM_SHARED((_N, W), jnp.float32),
            pltpu.SemaphoreType.DMA,
            pltpu.SemaphoreType.DMA,
            pltpu.SemaphoreType.DMA,
            pltpu.SemaphoreType.DMA,
            pltpu.SemaphoreType.DMA,
            pltpu.SemaphoreType.DMA,
            pltpu.SemaphoreType.DMA,
            pltpu.SemaphoreType.DMA,
        ],
    )
    def seg(table, src2d, dst2d, zeros, out, sidx, didx, rows0, rows1, rows2,
            rows3, acc, gs0, gs1, gs2, gs3, ss0, ss1, ss2, ss3):
        c = lax.axis_index("c")
        s = lax.axis_index("s")
        wid = c * _NS + s
        row0 = s * _RPT0
        @pl.when(s < 15)
        def _():
            pltpu.sync_copy(zeros.at[pl.ds(0, _RPT0)],
                            acc.at[pl.ds(row0, _RPT0)])

        @pl.when(s == 15)
        def _():
            pltpu.sync_copy(zeros.at[pl.ds(0, _RPT1)],
                            acc.at[pl.ds(row0, _RPT1)])

        pltpu.sync_copy(src2d.at[pl.ds(wid * _NCHP, _NCHP)], sidx)
        pltpu.sync_copy(dst2d.at[pl.ds(wid * _NCHP, _NCHP)], didx)
        plsc.subcore_barrier()

        bufs = (rows0, rows1, rows2, rows3)
        gsems = (gs0, gs1, gs2, gs3)
        ssems = (ss0, ss1, ss2, ss3)

        pltpu.async_copy(table.at[sidx.at[0]], bufs[0], gsems[0])
        pltpu.async_copy(table.at[sidx.at[1]], bufs[1], gsems[1])

        def body(i, carry):
            j = i + 2
            for b in range(4):
                @pl.when((i % 4) == b)
                def _(b=b):
                    pltpu.make_async_copy(table.at[sidx.at[i]], bufs[b],
                                          gsems[b]).wait()
                    pltpu.async_copy(bufs[b], acc.at[didx.at[i]], ssems[b],
                                     add=True)
            for b in range(4):
                cond = jnp.logical_and(j < _NCH, (j % 4) == b)

                @pl.when(jnp.logical_and(cond, i >= 2))
                def _(b=b):
                    pltpu.make_async_copy(bufs[b], acc.at[didx.at[0]],
                                          ssems[b]).wait()
                    pltpu.async_copy(table.at[sidx.at[j]], bufs[b], gsems[b])

                @pl.when(jnp.logical_and(cond, i < 2))
                def _(b=b):
                    pltpu.async_copy(table.at[sidx.at[j]], bufs[b], gsems[b])
            return carry

        lax.fori_loop(0, _NCH, body, 0)
        for b in range(4):
            pltpu.make_async_copy(bufs[b], acc.at[didx.at[0]], ssems[b]).wait()
        plsc.subcore_barrier()

        @pl.when(s < 15)
        def _():
            pltpu.sync_copy(acc.at[pl.ds(row0, _RPT0)],
                            out.at[c, pl.ds(row0, _RPT0)])

        @pl.when(s == 15)
        def _():
            pltpu.sync_copy(acc.at[pl.ds(row0, _RPT1)],
                            out.at[c, pl.ds(row0, _RPT1)])

    return seg


_segsum80 = _make_segsum(80)
_segsum64 = _make_segsum(64)


def _tc_a(x, wl1t, wr1t):
    def body(x_ref, wl_ref, wr_ref, y_ref, r_ref):
        xb = x_ref[...]
        y = jnp.dot(xb, wl_ref[...], preferred_element_type=jnp.float32)
        extra = (lax.broadcasted_iota(jnp.int32, (_BM, 16), 1) == 0)
        y_ref[...] = jnp.concatenate([y, extra.astype(jnp.float32)], axis=1)
        r_ref[...] = jnp.dot(xb, wr_ref[...], preferred_element_type=jnp.float32)

    return pl.pallas_call(
        body,
        grid=(_N // _BM,),
        in_specs=[
            pl.BlockSpec((_BM, 128), lambda i: (i, 0)),
            pl.BlockSpec((128, 64), lambda i: (0, 0)),
            pl.BlockSpec((128, 64), lambda i: (0, 0)),
        ],
        out_specs=[
            pl.BlockSpec((_BM, 80), lambda i: (i, 0)),
            pl.BlockSpec((_BM, 64), lambda i: (i, 0)),
        ],
        out_shape=[
            jax.ShapeDtypeStruct((_N, 80), jnp.float32),
            jax.ShapeDtypeStruct((_N, 64), jnp.float32),
        ],
    )(x, wl1t, wr1t)


def _tc_b(p1, bl1, r1):
    def body(p_ref, bl_ref, r_ref, h_ref, inv_ref):
        p = p_ref[0] + p_ref[1]
        lane = lax.broadcasted_iota(jnp.int32, (_BM, 80), 1)
        cnt = jnp.sum(jnp.where(lane == 64, p, 0.0), axis=1, keepdims=True)
        inv = 1.0 / jnp.maximum(cnt, 1.0)
        h = p[:, :64] * inv + bl_ref[...] + r_ref[...]
        h_ref[...] = jnp.maximum(h, 0.0)
        inv_ref[...] = inv

    return pl.pallas_call(
        body,
        grid=(_N // _BM,),
        in_specs=[
            pl.BlockSpec((_NC, _BM, 80), lambda i: (0, i, 0)),
            pl.BlockSpec((1, 64), lambda i: (0, 0)),
            pl.BlockSpec((_BM, 64), lambda i: (i, 0)),
        ],
        out_specs=[
            pl.BlockSpec((_BM, 64), lambda i: (i, 0)),
            pl.BlockSpec((_BM, 1), lambda i: (i, 0)),
        ],
        out_shape=[
            jax.ShapeDtypeStruct((_N, 64), jnp.float32),
            jax.ShapeDtypeStruct((_N, 1), jnp.float32),
        ],
    )(p1, bl1, r1)


def _tc_c(p2, inv, h1, wl2t, bl2, wr2t, wl3t, wr3t):
    def body(p_ref, inv_ref, h1_ref, wl2_ref, bl2_ref, wr2_ref, wl3_ref,
             wr3_ref, y3_ref, r3_ref):
        m2 = (p_ref[0] + p_ref[1]) * inv_ref[...]
        h2 = jnp.dot(m2, wl2_ref[...], preferred_element_type=jnp.float32)
        h2 = h2 + bl2_ref[...]
        h2 = h2 + jnp.dot(h1_ref[...], wr2_ref[...],
                          preferred_element_type=jnp.float32)
        h2 = jnp.maximum(h2, 0.0)
        y3_ref[...] = jnp.dot(h2, wl3_ref[...], preferred_element_type=jnp.float32)
        r3_ref[...] = jnp.dot(h2, wr3_ref[...], preferred_element_type=jnp.float32)

    return pl.pallas_call(
        body,
        grid=(_N // _BM,),
        in_specs=[
            pl.BlockSpec((_NC, _BM, 64), lambda i: (0, i, 0)),
            pl.BlockSpec((_BM, 1), lambda i: (i, 0)),
            pl.BlockSpec((_BM, 64), lambda i: (i, 0)),
            pl.BlockSpec((64, 128), lambda i: (0, 0)),
            pl.BlockSpec((1, 128), lambda i: (0, 0)),
            pl.BlockSpec((64, 128), lambda i: (0, 0)),
            pl.BlockSpec((128, 64), lambda i: (0, 0)),
            pl.BlockSpec((128, 64), lambda i: (0, 0)),
        ],
        out_specs=[
            pl.BlockSpec((_BM, 64), lambda i: (i, 0)),
            pl.BlockSpec((_BM, 64), lambda i: (i, 0)),
        ],
        out_shape=[
            jax.ShapeDtypeStruct((_N, 64), jnp.float32),
            jax.ShapeDtypeStruct((_N, 64), jnp.float32),
        ],
    )(p2, inv, h1, wl2t, bl2, wr2t, wl3t, wr3t)


def _tc_d(p3, inv, r3, bl3, w4t, b4):
    def body(p_ref, inv_ref, r_ref, bl_ref, w4_ref, b4_ref, o_ref):
        h3 = (p_ref[0] + p_ref[1]) * inv_ref[...] + bl_ref[...] + r_ref[...]
        h3 = jnp.maximum(h3, 0.0)
        o_ref[...] = jnp.dot(h3, w4_ref[...],
                             preferred_element_type=jnp.float32) + b4_ref[...]

    return pl.pallas_call(
        body,
        grid=(_N // _BM,),
        in_specs=[
            pl.BlockSpec((_NC, _BM, 64), lambda i: (0, i, 0)),
            pl.BlockSpec((_BM, 1), lambda i: (i, 0)),
            pl.BlockSpec((_BM, 64), lambda i: (i, 0)),
            pl.BlockSpec((1, 64), lambda i: (0, 0)),
            pl.BlockSpec((64, 128), lambda i: (0, 0)),
            pl.BlockSpec((1, 128), lambda i: (0, 0)),
        ],
        out_specs=pl.BlockSpec((_BM, 128), lambda i: (i, 0)),
        out_shape=jax.ShapeDtypeStruct((_N, 128), jnp.float32),
    )(p3, inv, r3, bl3, w4t, b4)


def kernel(x, edge_index, Wl1, bl1, Wr1, Wl2, bl2, Wr2, Wl3, bl3, Wr3, W4, b4):
    e4 = edge_index.reshape(2, _NW, _NCH, _CK)
    epad = jnp.concatenate(
        [e4, jnp.zeros((2, _NW, _NCHP - _NCH, _CK), jnp.int32)], axis=2
    ).reshape(2, _NW * _NCHP, _CK)
    src = epad[0]
    dst = epad[1]
    zeros80 = jnp.zeros((_RPT0, 80), jnp.float32)
    zeros64 = jnp.zeros((_RPT0, 64), jnp.float32)
    w4t = jnp.zeros((64, 128), jnp.float32).at[:, :3].set(W4.T)
    b4p = jnp.zeros((1, 128), jnp.float32).at[0, :3].set(b4)

    y1, r1 = _tc_a(x, Wl1.T, Wr1.T)
    p1 = _segsum80(y1, src, dst, zeros80)
    h1, inv = _tc_b(p1, bl1.reshape(1, 64), r1)
    p2 = _segsum64(h1, src, dst, zeros64)
    y3, r3 = _tc_c(p2, inv, h1, Wl2.T, bl2.reshape(1, 128), Wr2.T,
                   Wl3.T, Wr3.T)
    p3 = _segsum64(y3, src, dst, zeros64)
    out = _tc_d(p3, inv, r3, bl3.reshape(1, 64), w4t, b4p)
    return out[:, :3]


# speedup vs baseline: 15.7159x; 1.0994x over previous
import functools

import jax
import jax.numpy as jnp
from jax import lax
from jax.experimental import pallas as pl
from jax.experimental.pallas import tpu as pltpu
from jax.experimental.pallas import tpu_sc as plsc

_N = 10000
_E = 320000
_NC = 2
_NS = 16
_NW = _NC * _NS
_CK = 128
_EPW = _E // _NW
_NCH = 79
_NDUM = _NCH * _CK - _EPW
_NCHP = 80
_NSAC = 16
_RPT0 = 632
_RPT1 = _N - 15 * _RPT0
_BM = 1000


def _make_segsum(W):
    mesh = plsc.VectorSubcoreMesh(core_axis_name="c", subcore_axis_name="s")

    @functools.partial(
        pl.kernel,
        mesh=mesh,
        compiler_params=pltpu.CompilerParams(use_tc_tiling_on_sc=False),
        out_type=jax.ShapeDtypeStruct((_NC, _N, W), jnp.float32),
        scratch_types=[
            pltpu.VMEM((_NCHP, _CK), jnp.int32),
            pltpu.VMEM((_NCHP, _CK), jnp.int32),
            pltpu.VMEM((_CK, W), jnp.float32),
            pltpu.VMEM((_CK, W), jnp.float32),
            pltpu.VMEM((_CK, W), jnp.float32),
            pltpu.VMEM((_CK, W), jnp.float32),
            pltpu.VMEM_SHARED((_N + _NSAC, W), jnp.float32),
            pltpu.SemaphoreType.DMA,
            pltpu.SemaphoreType.DMA,
            pltpu.SemaphoreType.DMA,
            pltpu.SemaphoreType.DMA,
            pltpu.SemaphoreType.DMA,
            pltpu.SemaphoreType.DMA,
            pltpu.SemaphoreType.DMA,
            pltpu.SemaphoreType.DMA,
        ],
    )
    def seg(table, src2d, dst2d, zeros, out, sidx, didx, rows0, rows1, rows2,
            rows3, acc, gs0, gs1, gs2, gs3, ss0, ss1, ss2, ss3):
        c = lax.axis_index("c")
        s = lax.axis_index("s")
        wid = c * _NS + s
        row0 = s * _RPT0
        @pl.when(s < 15)
        def _():
            pltpu.sync_copy(zeros.at[pl.ds(row0, _RPT0)],
                            acc.at[pl.ds(row0, _RPT0)])

        @pl.when(s == 15)
        def _():
            pltpu.sync_copy(zeros.at[pl.ds(row0, _RPT1)],
                            acc.at[pl.ds(row0, _RPT1)])

        pltpu.sync_copy(src2d.at[pl.ds(wid * _NCHP, _NCHP)], sidx)
        pltpu.sync_copy(dst2d.at[pl.ds(wid * _NCHP, _NCHP)], didx)
        plsc.subcore_barrier()

        bufs = (rows0, rows1, rows2, rows3)
        gsems = (gs0, gs1, gs2, gs3)
        ssems = (ss0, ss1, ss2, ss3)

        pltpu.async_copy(table.at[sidx.at[0]], bufs[0], gsems[0])
        pltpu.async_copy(table.at[sidx.at[1]], bufs[1], gsems[1])

        def body(i, carry):
            j = i + 2
            for b in range(4):
                @pl.when((i % 4) == b)
                def _(b=b):
                    pltpu.make_async_copy(table.at[sidx.at[i]], bufs[b],
                                          gsems[b]).wait()
                    pltpu.async_copy(bufs[b], acc.at[didx.at[i]], ssems[b],
                                     add=True)
            for b in range(4):
                cond = jnp.logical_and(j < _NCH, (j % 4) == b)

                @pl.when(jnp.logical_and(cond, i >= 2))
                def _(b=b):
                    pltpu.make_async_copy(bufs[b], acc.at[didx.at[0]],
                                          ssems[b]).wait()
                    pltpu.async_copy(table.at[sidx.at[j]], bufs[b], gsems[b])

                @pl.when(jnp.logical_and(cond, i < 2))
                def _(b=b):
                    pltpu.async_copy(table.at[sidx.at[j]], bufs[b], gsems[b])
            return carry

        lax.fori_loop(0, _NCH, body, 0)
        for b in range(4):
            pltpu.make_async_copy(bufs[b], acc.at[didx.at[0]], ssems[b]).wait()
        plsc.subcore_barrier()

        @pl.when(s < 15)
        def _():
            pltpu.sync_copy(acc.at[pl.ds(row0, _RPT0)],
                            out.at[c, pl.ds(row0, _RPT0)])

        @pl.when(s == 15)
        def _():
            pltpu.sync_copy(acc.at[pl.ds(row0, _RPT1)],
                            out.at[c, pl.ds(row0, _RPT1)])

    return seg


_segsum80 = _make_segsum(80)
_segsum64 = _make_segsum(64)


def _tc_a(x, wl1t, wr1t):
    def body(x_ref, wl_ref, wr_ref, y_ref, r_ref):
        xb = x_ref[...]
        y = jnp.dot(xb, wl_ref[...], preferred_element_type=jnp.float32)
        extra = (lax.broadcasted_iota(jnp.int32, (_BM, 16), 1) == 0)
        y_ref[...] = jnp.concatenate([y, extra.astype(jnp.float32)], axis=1)
        r_ref[...] = jnp.dot(xb, wr_ref[...], preferred_element_type=jnp.float32)

    return pl.pallas_call(
        body,
        grid=(_N // _BM,),
        in_specs=[
            pl.BlockSpec((_BM, 128), lambda i: (i, 0)),
            pl.BlockSpec((128, 64), lambda i: (0, 0)),
            pl.BlockSpec((128, 64), lambda i: (0, 0)),
        ],
        out_specs=[
            pl.BlockSpec((_BM, 80), lambda i: (i, 0)),
            pl.BlockSpec((_BM, 64), lambda i: (i, 0)),
        ],
        out_shape=[
            jax.ShapeDtypeStruct((_N, 80), jnp.float32),
            jax.ShapeDtypeStruct((_N, 64), jnp.float32),
        ],
    )(x, wl1t, wr1t)


def _tc_b(p1, bl1, r1):
    def body(p_ref, bl_ref, r_ref, h_ref, inv_ref):
        p = p_ref[0] + p_ref[1]
        lane = lax.broadcasted_iota(jnp.int32, (_BM, 80), 1)
        cnt = jnp.sum(jnp.where(lane == 64, p, 0.0), axis=1, keepdims=True)
        inv = 1.0 / jnp.maximum(cnt, 1.0)
        h = p[:, :64] * inv + bl_ref[...] + r_ref[...]
        h_ref[...] = jnp.maximum(h, 0.0)
        inv_ref[...] = inv

    return pl.pallas_call(
        body,
        grid=(_N // _BM,),
        in_specs=[
            pl.BlockSpec((_NC, _BM, 80), lambda i: (0, i, 0)),
            pl.BlockSpec((1, 64), lambda i: (0, 0)),
            pl.BlockSpec((_BM, 64), lambda i: (i, 0)),
        ],
        out_specs=[
            pl.BlockSpec((_BM, 64), lambda i: (i, 0)),
            pl.BlockSpec((_BM, 1), lambda i: (i, 0)),
        ],
        out_shape=[
            jax.ShapeDtypeStruct((_N, 64), jnp.float32),
            jax.ShapeDtypeStruct((_N, 1), jnp.float32),
        ],
    )(p1, bl1, r1)


def _tc_c(p2, inv, h1, wl2t, bl2, wr2t, wl3t, wr3t):
    def body(p_ref, inv_ref, h1_ref, wl2_ref, bl2_ref, wr2_ref, wl3_ref,
             wr3_ref, y3_ref, r3_ref):
        m2 = (p_ref[0] + p_ref[1]) * inv_ref[...]
        h2 = jnp.dot(m2, wl2_ref[...], preferred_element_type=jnp.float32)
        h2 = h2 + bl2_ref[...]
        h2 = h2 + jnp.dot(h1_ref[...], wr2_ref[...],
                          preferred_element_type=jnp.float32)
        h2 = jnp.maximum(h2, 0.0)
        y3_ref[...] = jnp.dot(h2, wl3_ref[...], preferred_element_type=jnp.float32)
        r3_ref[...] = jnp.dot(h2, wr3_ref[...], preferred_element_type=jnp.float32)

    return pl.pallas_call(
        body,
        grid=(_N // _BM,),
        in_specs=[
            pl.BlockSpec((_NC, _BM, 64), lambda i: (0, i, 0)),
            pl.BlockSpec((_BM, 1), lambda i: (i, 0)),
            pl.BlockSpec((_BM, 64), lambda i: (i, 0)),
            pl.BlockSpec((64, 128), lambda i: (0, 0)),
            pl.BlockSpec((1, 128), lambda i: (0, 0)),
            pl.BlockSpec((64, 128), lambda i: (0, 0)),
            pl.BlockSpec((128, 64), lambda i: (0, 0)),
            pl.BlockSpec((128, 64), lambda i: (0, 0)),
        ],
        out_specs=[
            pl.BlockSpec((_BM, 64), lambda i: (i, 0)),
            pl.BlockSpec((_BM, 64), lambda i: (i, 0)),
        ],
        out_shape=[
            jax.ShapeDtypeStruct((_N, 64), jnp.float32),
            jax.ShapeDtypeStruct((_N, 64), jnp.float32),
        ],
    )(p2, inv, h1, wl2t, bl2, wr2t, wl3t, wr3t)


def _tc_d(p3, inv, r3, bl3, w4t, b4):
    def body(p_ref, inv_ref, r_ref, bl_ref, w4_ref, b4_ref, o_ref):
        h3 = (p_ref[0] + p_ref[1]) * inv_ref[...] + bl_ref[...] + r_ref[...]
        h3 = jnp.maximum(h3, 0.0)
        o_ref[...] = jnp.dot(h3, w4_ref[...],
                             preferred_element_type=jnp.float32) + b4_ref[...]

    return pl.pallas_call(
        body,
        grid=(_N // _BM,),
        in_specs=[
            pl.BlockSpec((_NC, _BM, 64), lambda i: (0, i, 0)),
            pl.BlockSpec((_BM, 1), lambda i: (i, 0)),
            pl.BlockSpec((_BM, 64), lambda i: (i, 0)),
            pl.BlockSpec((1, 64), lambda i: (0, 0)),
            pl.BlockSpec((64, 128), lambda i: (0, 0)),
            pl.BlockSpec((1, 128), lambda i: (0, 0)),
        ],
        out_specs=pl.BlockSpec((_BM, 128), lambda i: (i, 0)),
        out_shape=jax.ShapeDtypeStruct((_N, 128), jnp.float32),
    )(p3, inv, r3, bl3, w4t, b4)


def kernel(x, edge_index, Wl1, bl1, Wr1, Wl2, bl2, Wr2, Wl3, bl3, Wr3, W4, b4):
    e3 = edge_index.reshape(2, _NW, _EPW)
    ar = jnp.arange(_NDUM, dtype=jnp.int32)
    dsrc = jnp.broadcast_to((ar * 89) % _N, (_NW, _NDUM))
    ddst = jnp.broadcast_to(_N + (ar % _NSAC), (_NW, _NDUM))
    dummy = jnp.stack([dsrc, ddst])
    slab = jnp.concatenate([e3, dummy], axis=2)
    slab = slab.reshape(2, _NW, _NCH, _CK)
    slab = jnp.concatenate(
        [slab, jnp.zeros((2, _NW, _NCHP - _NCH, _CK), jnp.int32)], axis=2)
    epad = slab.reshape(2, _NW * _NCHP, _CK)
    src = epad[0]
    dst = epad[1]
    zeros80 = jnp.zeros((_N, 80), jnp.float32)
    zeros64 = jnp.zeros((_N, 64), jnp.float32)
    w4t = jnp.zeros((64, 128), jnp.float32).at[:, :3].set(W4.T)
    b4p = jnp.zeros((1, 128), jnp.float32).at[0, :3].set(b4)

    y1, r1 = _tc_a(x, Wl1.T, Wr1.T)
    p1 = _segsum80(y1, src, dst, zeros80)
    h1, inv = _tc_b(p1, bl1.reshape(1, 64), r1)
    p2 = _segsum64(h1, src, dst, zeros64)
    y3, r3 = _tc_c(p2, inv, h1, Wl2.T, bl2.reshape(1, 128), Wr2.T,
                   Wl3.T, Wr3.T)
    p3 = _segsum64(y3, src, dst, zeros64)
    out = _tc_d(p3, inv, r3, bl3.reshape(1, 64), w4t, b4p)
    return out[:, :3]


# speedup vs baseline: 15.8207x; 1.0067x over previous
import functools

import jax
import jax.numpy as jnp
from jax import lax
from jax.experimental import pallas as pl
from jax.experimental.pallas import tpu as pltpu
from jax.experimental.pallas import tpu_sc as plsc

_N = 10000
_E = 320000
_NC = 2
_NS = 16
_NW = _NC * _NS
_CK = 128
_EPW = _E // _NW
_NCH = 79
_NDUM = _NCH * _CK - _EPW
_NCHP = 80
_NSAC = 16
_RPT0 = 632
_RPT1 = _N - 15 * _RPT0
_CW = 32
_BM = 2000


_mesh = plsc.VectorSubcoreMesh(core_axis_name="c", subcore_axis_name="s")


@functools.partial(
    pl.kernel,
    mesh=_mesh,
    compiler_params=pltpu.CompilerParams(use_tc_tiling_on_sc=False),
    out_type=jax.ShapeDtypeStruct((_NC, _N, 64), jnp.float32),
    scratch_types=[
        pltpu.VMEM((_NCHP, _CK), jnp.int32),
        pltpu.VMEM((_NCHP, _CK), jnp.int32),
        pltpu.VMEM((_CK, 64), jnp.float32),
        pltpu.VMEM((_CK, 64), jnp.float32),
        pltpu.VMEM((_CK, 64), jnp.float32),
        pltpu.VMEM((_CK, 64), jnp.float32),
        pltpu.VMEM_SHARED((_N + _NSAC, 64), jnp.float32),
        pltpu.SemaphoreType.DMA,
        pltpu.SemaphoreType.DMA,
        pltpu.SemaphoreType.DMA,
        pltpu.SemaphoreType.DMA,
        pltpu.SemaphoreType.DMA,
        pltpu.SemaphoreType.DMA,
        pltpu.SemaphoreType.DMA,
        pltpu.SemaphoreType.DMA,
    ],
)
def _segsum(table, src2d, dst2d, zeros, out, sidx, didx, rows0, rows1, rows2,
            rows3, acc, gs0, gs1, gs2, gs3, ss0, ss1, ss2, ss3):
    c = lax.axis_index("c")
    s = lax.axis_index("s")
    wid = c * _NS + s
    row0 = s * _RPT0

    @pl.when(s < 15)
    def _():
        pltpu.sync_copy(zeros.at[pl.ds(row0, _RPT0)],
                        acc.at[pl.ds(row0, _RPT0)])

    @pl.when(s == 15)
    def _():
        pltpu.sync_copy(zeros.at[pl.ds(row0, _RPT1)],
                        acc.at[pl.ds(row0, _RPT1)])

    pltpu.sync_copy(src2d.at[pl.ds(wid * _NCHP, _NCHP)], sidx)
    pltpu.sync_copy(dst2d.at[pl.ds(wid * _NCHP, _NCHP)], didx)
    plsc.subcore_barrier()

    bufs = (rows0, rows1, rows2, rows3)
    gsems = (gs0, gs1, gs2, gs3)
    ssems = (ss0, ss1, ss2, ss3)

    pltpu.async_copy(table.at[sidx.at[0]], bufs[0], gsems[0])
    pltpu.async_copy(table.at[sidx.at[1]], bufs[1], gsems[1])

    def body(i, carry):
        j = i + 2
        for b in range(4):
            @pl.when((i % 4) == b)
            def _(b=b):
                pltpu.make_async_copy(table.at[sidx.at[i]], bufs[b],
                                      gsems[b]).wait()
                pltpu.async_copy(bufs[b], acc.at[didx.at[i]], ssems[b],
                                 add=True)
        for b in range(4):
            cond = jnp.logical_and(j < _NCH, (j % 4) == b)

            @pl.when(jnp.logical_and(cond, i >= 2))
            def _(b=b):
                pltpu.make_async_copy(bufs[b], acc.at[didx.at[0]],
                                      ssems[b]).wait()
                pltpu.async_copy(table.at[sidx.at[j]], bufs[b], gsems[b])

            @pl.when(jnp.logical_and(cond, i < 2))
            def _(b=b):
                pltpu.async_copy(table.at[sidx.at[j]], bufs[b], gsems[b])
        return carry

    lax.fori_loop(0, _NCH, body, 0)
    for b in range(4):
        pltpu.make_async_copy(bufs[b], acc.at[didx.at[0]], ssems[b]).wait()
    plsc.subcore_barrier()

    @pl.when(s < 15)
    def _():
        pltpu.sync_copy(acc.at[pl.ds(row0, _RPT0)],
                        out.at[c, pl.ds(row0, _RPT0)])

    @pl.when(s == 15)
    def _():
        pltpu.sync_copy(acc.at[pl.ds(row0, _RPT1)],
                        out.at[c, pl.ds(row0, _RPT1)])


@functools.partial(
    pl.kernel,
    mesh=_mesh,
    compiler_params=pltpu.CompilerParams(use_tc_tiling_on_sc=False),
    out_type=jax.ShapeDtypeStruct((_NC, _N, _CW), jnp.float32),
    scratch_types=[
        pltpu.VMEM((_NCHP, _CK), jnp.int32),
        pltpu.VMEM((_CK, _CW), jnp.float32),
        pltpu.VMEM_SHARED((_N + _NSAC, _CW), jnp.float32),
        pltpu.SemaphoreType.DMA,
        pltpu.SemaphoreType.DMA,
        pltpu.SemaphoreType.DMA,
        pltpu.SemaphoreType.DMA,
    ],
)
def _degree(dst2d, zeros, out, didx, ones, acc, s0, s1, s2, s3):
    c = lax.axis_index("c")
    s = lax.axis_index("s")
    wid = c * _NS + s
    row0 = s * _RPT0

    @pl.when(s < 15)
    def _():
        pltpu.sync_copy(zeros.at[pl.ds(row0, _RPT0)],
                        acc.at[pl.ds(row0, _RPT0)])

    @pl.when(s == 15)
    def _():
        pltpu.sync_copy(zeros.at[pl.ds(row0, _RPT1)],
                        acc.at[pl.ds(row0, _RPT1)])

    pltpu.sync_copy(dst2d.at[pl.ds(wid * _NCHP, _NCHP)], didx)

    def fill(i, carry):
        for j in range(_CW // 16):
            ones[i, pl.ds(j * 16, 16)] = jnp.full((16,), 1.0, jnp.float32)
        return carry

    lax.fori_loop(0, _CK, fill, 0)
    plsc.subcore_barrier()

    sems = (s0, s1, s2, s3)

    def body(i, carry):
        for b in range(4):
            @pl.when(jnp.logical_and((i % 4) == b, i >= 4))
            def _(b=b):
                pltpu.make_async_copy(ones, acc.at[didx.at[0]],
                                      sems[b]).wait()
            @pl.when((i % 4) == b)
            def _(b=b):
                pltpu.async_copy(ones, acc.at[didx.at[i]], sems[b], add=True)
        return carry

    lax.fori_loop(0, _NCH, body, 0)
    for b in range(4):
        pltpu.make_async_copy(ones, acc.at[didx.at[0]], sems[b]).wait()
    plsc.subcore_barrier()

    @pl.when(s < 15)
    def _():
        pltpu.sync_copy(acc.at[pl.ds(row0, _RPT0)],
                        out.at[c, pl.ds(row0, _RPT0)])

    @pl.when(s == 15)
    def _():
        pltpu.sync_copy(acc.at[pl.ds(row0, _RPT1)],
                        out.at[c, pl.ds(row0, _RPT1)])


def _tc_a(x, wl1t, wr1t):
    def body(x_ref, wl_ref, wr_ref, y_ref, r_ref):
        xb = x_ref[...]
        y_ref[...] = jnp.dot(xb, wl_ref[...], preferred_element_type=jnp.float32)
        r_ref[...] = jnp.dot(xb, wr_ref[...], preferred_element_type=jnp.float32)

    return pl.pallas_call(
        body,
        grid=(_N // _BM,),
        in_specs=[
            pl.BlockSpec((_BM, 128), lambda i: (i, 0)),
            pl.BlockSpec((128, 64), lambda i: (0, 0)),
            pl.BlockSpec((128, 64), lambda i: (0, 0)),
        ],
        out_specs=[
            pl.BlockSpec((_BM, 64), lambda i: (i, 0)),
            pl.BlockSpec((_BM, 64), lambda i: (i, 0)),
        ],
        out_shape=[
            jax.ShapeDtypeStruct((_N, 64), jnp.float32),
            jax.ShapeDtypeStruct((_N, 64), jnp.float32),
        ],
    )(x, wl1t, wr1t)


def _inv_from_counts(pc_ref):
    pc = pc_ref[0] + pc_ref[1]
    return 1.0 / jnp.maximum(pc[:, :1], 1.0)


def _tc_b(p1, pc, r1, bl1):
    def body(p_ref, pc_ref, r_ref, bl_ref, h_ref):
        p = p_ref[0] + p_ref[1]
        inv = _inv_from_counts(pc_ref)
        h_ref[...] = jnp.maximum(p * inv + bl_ref[...] + r_ref[...], 0.0)

    return pl.pallas_call(
        body,
        grid=(_N // _BM,),
        in_specs=[
            pl.BlockSpec((_NC, _BM, 64), lambda i: (0, i, 0)),
            pl.BlockSpec((_NC, _BM, _CW), lambda i: (0, i, 0)),
            pl.BlockSpec((_BM, 64), lambda i: (i, 0)),
            pl.BlockSpec((1, 64), lambda i: (0, 0)),
        ],
        out_specs=pl.BlockSpec((_BM, 64), lambda i: (i, 0)),
        out_shape=jax.ShapeDtypeStruct((_N, 64), jnp.float32),
    )(p1, pc, r1, bl1)


def _tc_c(p2, pc, h1, wl2t, bl2, wr2t, wl3t, wr3t):
    def body(p_ref, pc_ref, h1_ref, wl2_ref, bl2_ref, wr2_ref, wl3_ref,
             wr3_ref, y3_ref, r3_ref):
        m2 = p_ref[0] + p_ref[1]
        inv = _inv_from_counts(pc_ref)
        m2 = m2 * inv
        h1 = h1_ref[...]
        h2 = jnp.dot(m2, wl2_ref[...], preferred_element_type=jnp.float32)
        h2 = h2 + bl2_ref[...]
        h2 = h2 + jnp.dot(h1, wr2_ref[...], preferred_element_type=jnp.float32)
        h2 = jnp.maximum(h2, 0.0)
        y3_ref[...] = jnp.dot(h2, wl3_ref[...], preferred_element_type=jnp.float32)
        r3_ref[...] = jnp.dot(h2, wr3_ref[...], preferred_element_type=jnp.float32)

    return pl.pallas_call(
        body,
        grid=(_N // _BM,),
        in_specs=[
            pl.BlockSpec((_NC, _BM, 64), lambda i: (0, i, 0)),
            pl.BlockSpec((_NC, _BM, _CW), lambda i: (0, i, 0)),
            pl.BlockSpec((_BM, 64), lambda i: (i, 0)),
            pl.BlockSpec((64, 128), lambda i: (0, 0)),
            pl.BlockSpec((1, 128), lambda i: (0, 0)),
            pl.BlockSpec((64, 128), lambda i: (0, 0)),
            pl.BlockSpec((128, 64), lambda i: (0, 0)),
            pl.BlockSpec((128, 64), lambda i: (0, 0)),
        ],
        out_specs=[
            pl.BlockSpec((_BM, 64), lambda i: (i, 0)),
            pl.BlockSpec((_BM, 64), lambda i: (i, 0)),
        ],
        out_shape=[
            jax.ShapeDtypeStruct((_N, 64), jnp.float32),
            jax.ShapeDtypeStruct((_N, 64), jnp.float32),
        ],
    )(p2, pc, h1, wl2t, bl2, wr2t, wl3t, wr3t)


def _tc_d(p3, pc, r3, bl3, w4t, b4):
    def body(p_ref, pc_ref, r_ref, bl_ref, w4_ref, b4_ref, o_ref):
        m3 = p_ref[0] + p_ref[1]
        inv = _inv_from_counts(pc_ref)
        h3 = jnp.maximum(m3 * inv + bl_ref[...] + r_ref[...], 0.0)
        o_ref[...] = jnp.dot(h3, w4_ref[...],
                             preferred_element_type=jnp.float32) + b4_ref[...]

    return pl.pallas_call(
        body,
        grid=(_N // _BM,),
        in_specs=[
            pl.BlockSpec((_NC, _BM, 64), lambda i: (0, i, 0)),
            pl.BlockSpec((_NC, _BM, _CW), lambda i: (0, i, 0)),
            pl.BlockSpec((_BM, 64), lambda i: (i, 0)),
            pl.BlockSpec((1, 64), lambda i: (0, 0)),
            pl.BlockSpec((64, 128), lambda i: (0, 0)),
            pl.BlockSpec((1, 128), lambda i: (0, 0)),
        ],
        out_specs=pl.BlockSpec((_BM, 128), lambda i: (i, 0)),
        out_shape=jax.ShapeDtypeStruct((_N, 128), jnp.float32),
    )(p3, pc, r3, bl3, w4t, b4)


def kernel(x, edge_index, Wl1, bl1, Wr1, Wl2, bl2, Wr2, Wl3, bl3, Wr3, W4, b4):
    e3 = edge_index.reshape(2, _NW, _EPW)
    ar = jnp.arange(_NDUM, dtype=jnp.int32)
    dsrc = jnp.broadcast_to((ar * 89) % _N, (_NW, _NDUM))
    ddst = jnp.broadcast_to(_N + (ar % _NSAC), (_NW, _NDUM))
    dummy = jnp.stack([dsrc, ddst])
    slab = jnp.concatenate([e3, dummy], axis=2)
    slab = slab.reshape(2, _NW, _NCH, _CK)
    slab = jnp.concatenate(
        [slab, jnp.zeros((2, _NW, _NCHP - _NCH, _CK), jnp.int32)], axis=2)
    epad = slab.reshape(2, _NW * _NCHP, _CK)
    src = epad[0]
    dst = epad[1]
    zeros64 = jnp.zeros((_N, 64), jnp.float32)
    zerosc = jnp.zeros((_N, _CW), jnp.float32)
    w4t = jnp.zeros((64, 128), jnp.float32).at[:, :3].set(W4.T)
    b4p = jnp.zeros((1, 128), jnp.float32).at[0, :3].set(b4)

    pc = _degree(dst, zerosc)
    y1, r1 = _tc_a(x, Wl1.T, Wr1.T)
    p1 = _segsum(y1, src, dst, zeros64)
    h1 = _tc_b(p1, pc, r1, bl1.reshape(1, 64))
    p2 = _segsum(h1, src, dst, zeros64)
    y3, r3 = _tc_c(p2, pc, h1, Wl2.T, bl2.reshape(1, 128), Wr2.T, Wl3.T,
                   Wr3.T)
    p3 = _segsum(y3, src, dst, zeros64)
    out = _tc_d(p3, pc, r3, bl3.reshape(1, 64), w4t, b4p)
    return out[:, :3]


# speedup vs baseline: 16.1140x; 1.0185x over previous
import functools

import jax
import jax.numpy as jnp
from jax import lax
from jax.experimental import pallas as pl
from jax.experimental.pallas import tpu as pltpu
from jax.experimental.pallas import tpu_sc as plsc

_N = 10000
_E = 320000
_NC = 2
_NS = 16
_NW = _NC * _NS
_CK = 128
_EPW = _E // _NW
_NCH = 79
_NDUM = _NCH * _CK - _EPW
_NCHP = 80
_NSAC = 16
_RPT0 = 632
_RPT1 = _N - 15 * _RPT0
_CW = 32
_BM = 2000


_mesh = plsc.VectorSubcoreMesh(core_axis_name="c", subcore_axis_name="s")


@functools.partial(
    pl.kernel,
    mesh=_mesh,
    compiler_params=pltpu.CompilerParams(use_tc_tiling_on_sc=False),
    out_type=jax.ShapeDtypeStruct((_NC, _N, 64), jnp.float32),
    scratch_types=[
        pltpu.VMEM((_NCHP, _CK), jnp.int32),
        pltpu.VMEM((_NCHP, _CK), jnp.int32),
        pltpu.VMEM((_CK, 64), jnp.float32),
        pltpu.VMEM((_CK, 64), jnp.float32),
        pltpu.VMEM((_CK, 64), jnp.float32),
        pltpu.VMEM((_CK, 64), jnp.float32),
        pltpu.VMEM_SHARED((_N + _NSAC, 64), jnp.float32),
        pltpu.SemaphoreType.DMA,
        pltpu.SemaphoreType.DMA,
        pltpu.SemaphoreType.DMA,
        pltpu.SemaphoreType.DMA,
        pltpu.SemaphoreType.DMA,
        pltpu.SemaphoreType.DMA,
        pltpu.SemaphoreType.DMA,
        pltpu.SemaphoreType.DMA,
    ],
)
def _segsum(table, src2d, dst2d, zeros, out, sidx, didx, rows0, rows1, rows2,
            rows3, acc, gs0, gs1, gs2, gs3, ss0, ss1, ss2, ss3):
    c = lax.axis_index("c")
    s = lax.axis_index("s")
    wid = c * _NS + s
    row0 = s * _RPT0

    @pl.when(s < 15)
    def _():
        pltpu.sync_copy(zeros.at[pl.ds(row0, _RPT0)],
                        acc.at[pl.ds(row0, _RPT0)])

    @pl.when(s == 15)
    def _():
        pltpu.sync_copy(zeros.at[pl.ds(row0, _RPT1)],
                        acc.at[pl.ds(row0, _RPT1)])

    pltpu.sync_copy(src2d.at[pl.ds(wid * _NCHP, _NCHP)], sidx)
    pltpu.sync_copy(dst2d.at[pl.ds(wid * _NCHP, _NCHP)], didx)
    plsc.subcore_barrier()

    bufs = (rows0, rows1, rows2, rows3)
    gsems = (gs0, gs1, gs2, gs3)
    ssems = (ss0, ss1, ss2, ss3)

    pltpu.async_copy(table.at[sidx.at[0]], bufs[0], gsems[0])
    pltpu.async_copy(table.at[sidx.at[1]], bufs[1], gsems[1])

    def body(i, carry):
        j = i + 2
        for b in range(4):
            @pl.when((i % 4) == b)
            def _(b=b):
                pltpu.make_async_copy(table.at[sidx.at[i]], bufs[b],
                                      gsems[b]).wait()
                pltpu.async_copy(bufs[b], acc.at[didx.at[i]], ssems[b],
                                 add=True)
        for b in range(4):
            cond = jnp.logical_and(j < _NCH, (j % 4) == b)

            @pl.when(jnp.logical_and(cond, i >= 2))
            def _(b=b):
                pltpu.make_async_copy(bufs[b], acc.at[didx.at[0]],
                                      ssems[b]).wait()
                pltpu.async_copy(table.at[sidx.at[j]], bufs[b], gsems[b])

            @pl.when(jnp.logical_and(cond, i < 2))
            def _(b=b):
                pltpu.async_copy(table.at[sidx.at[j]], bufs[b], gsems[b])
        return carry

    lax.fori_loop(0, _NCH, body, 0)
    for b in range(4):
        pltpu.make_async_copy(bufs[b], acc.at[didx.at[0]], ssems[b]).wait()
    plsc.subcore_barrier()

    @pl.when(s < 15)
    def _():
        pltpu.sync_copy(acc.at[pl.ds(row0, _RPT0)],
                        out.at[c, pl.ds(row0, _RPT0)])

    @pl.when(s == 15)
    def _():
        pltpu.sync_copy(acc.at[pl.ds(row0, _RPT1)],
                        out.at[c, pl.ds(row0, _RPT1)])


@functools.partial(
    pl.kernel,
    mesh=_mesh,
    compiler_params=pltpu.CompilerParams(use_tc_tiling_on_sc=False),
    out_type=jax.ShapeDtypeStruct((_NC, _N, _CW), jnp.float32),
    scratch_types=[
        pltpu.VMEM((_NCHP, _CK), jnp.int32),
        pltpu.VMEM((_CK, _CW), jnp.float32),
        pltpu.VMEM_SHARED((_N + _NSAC, _CW), jnp.float32),
        pltpu.SemaphoreType.DMA,
        pltpu.SemaphoreType.DMA,
        pltpu.SemaphoreType.DMA,
        pltpu.SemaphoreType.DMA,
    ],
)
def _degree(dst2d, zeros, out, didx, ones, acc, s0, s1, s2, s3):
    c = lax.axis_index("c")
    s = lax.axis_index("s")
    wid = c * _NS + s
    row0 = s * _RPT0

    @pl.when(s < 15)
    def _():
        pltpu.sync_copy(zeros.at[pl.ds(row0, _RPT0)],
                        acc.at[pl.ds(row0, _RPT0)])

    @pl.when(s == 15)
    def _():
        pltpu.sync_copy(zeros.at[pl.ds(row0, _RPT1)],
                        acc.at[pl.ds(row0, _RPT1)])

    pltpu.sync_copy(dst2d.at[pl.ds(wid * _NCHP, _NCHP)], didx)

    def fill(i, carry):
        for j in range(_CW // 16):
            ones[i, pl.ds(j * 16, 16)] = jnp.full((16,), 1.0, jnp.float32)
        return carry

    lax.fori_loop(0, _CK, fill, 0)
    plsc.subcore_barrier()

    sems = (s0, s1, s2, s3)

    def body(i, carry):
        for b in range(4):
            @pl.when(jnp.logical_and((i % 4) == b, i >= 4))
            def _(b=b):
                pltpu.make_async_copy(ones, acc.at[didx.at[0]],
                                      sems[b]).wait()
            @pl.when((i % 4) == b)
            def _(b=b):
                pltpu.async_copy(ones, acc.at[didx.at[i]], sems[b], add=True)
        return carry

    lax.fori_loop(0, _NCH, body, 0)
    for b in range(4):
        pltpu.make_async_copy(ones, acc.at[didx.at[0]], sems[b]).wait()
    plsc.subcore_barrier()

    @pl.when(s < 15)
    def _():
        pltpu.sync_copy(acc.at[pl.ds(row0, _RPT0)],
                        out.at[c, pl.ds(row0, _RPT0)])

    @pl.when(s == 15)
    def _():
        pltpu.sync_copy(acc.at[pl.ds(row0, _RPT1)],
                        out.at[c, pl.ds(row0, _RPT1)])


def _tc_a(x, wl1t, wr1t):
    def body(x_ref, wl_ref, wr_ref, y_ref, r_ref):
        xb = x_ref[...]
        y_ref[...] = jnp.dot(xb, wl_ref[...], preferred_element_type=jnp.float32)
        r_ref[...] = jnp.dot(xb, wr_ref[...], preferred_element_type=jnp.float32)

    return pl.pallas_call(
        body,
        grid=(_N // _BM,),
        in_specs=[
            pl.BlockSpec((_BM, 128), lambda i: (i, 0)),
            pl.BlockSpec((128, 64), lambda i: (0, 0)),
            pl.BlockSpec((128, 64), lambda i: (0, 0)),
        ],
        out_specs=[
            pl.BlockSpec((_BM, 64), lambda i: (i, 0)),
            pl.BlockSpec((_BM, 64), lambda i: (i, 0)),
        ],
        out_shape=[
            jax.ShapeDtypeStruct((_N, 64), jnp.float32),
            jax.ShapeDtypeStruct((_N, 64), jnp.float32),
        ],
    )(x, wl1t, wr1t)


def _inv_from_counts(pc_ref):
    pc = pc_ref[0] + pc_ref[1]
    return 1.0 / jnp.maximum(pc[:, :1], 1.0)


def _tc_b(p1, pc, r1, bl1):
    def body(p_ref, pc_ref, r_ref, bl_ref, h_ref):
        p = p_ref[0] + p_ref[1]
        inv = _inv_from_counts(pc_ref)
        h_ref[...] = jnp.maximum(p * inv + bl_ref[...] + r_ref[...], 0.0)

    return pl.pallas_call(
        body,
        grid=(_N // _BM,),
        in_specs=[
            pl.BlockSpec((_NC, _BM, 64), lambda i: (0, i, 0)),
            pl.BlockSpec((_NC, _BM, _CW), lambda i: (0, i, 0)),
            pl.BlockSpec((_BM, 64), lambda i: (i, 0)),
            pl.BlockSpec((1, 64), lambda i: (0, 0)),
        ],
        out_specs=pl.BlockSpec((_BM, 64), lambda i: (i, 0)),
        out_shape=jax.ShapeDtypeStruct((_N, 64), jnp.float32),
    )(p1, pc, r1, bl1)


def _tc_c(p2, pc, h1, wl2t, bl2, wr2t, wl3t, wr3t):
    def body(p_ref, pc_ref, h1_ref, wl2_ref, bl2_ref, wr2_ref, wl3_ref,
             wr3_ref, y3_ref, r3_ref):
        m2 = p_ref[0] + p_ref[1]
        inv = _inv_from_counts(pc_ref)
        m2 = m2 * inv
        h1 = h1_ref[...]
        h2 = jnp.dot(m2, wl2_ref[...], preferred_element_type=jnp.float32)
        h2 = h2 + bl2_ref[...]
        h2 = h2 + jnp.dot(h1, wr2_ref[...], preferred_element_type=jnp.float32)
        h2 = jnp.maximum(h2, 0.0)
        y3_ref[...] = jnp.dot(h2, wl3_ref[...], preferred_element_type=jnp.float32)
        r3_ref[...] = jnp.dot(h2, wr3_ref[...], preferred_element_type=jnp.float32)

    return pl.pallas_call(
        body,
        grid=(_N // _BM,),
        in_specs=[
            pl.BlockSpec((_NC, _BM, 64), lambda i: (0, i, 0)),
            pl.BlockSpec((_NC, _BM, _CW), lambda i: (0, i, 0)),
            pl.BlockSpec((_BM, 64), lambda i: (i, 0)),
            pl.BlockSpec((64, 128), lambda i: (0, 0)),
            pl.BlockSpec((1, 128), lambda i: (0, 0)),
            pl.BlockSpec((64, 128), lambda i: (0, 0)),
            pl.BlockSpec((128, 64), lambda i: (0, 0)),
            pl.BlockSpec((128, 64), lambda i: (0, 0)),
        ],
        out_specs=[
            pl.BlockSpec((_BM, 64), lambda i: (i, 0)),
            pl.BlockSpec((_BM, 64), lambda i: (i, 0)),
        ],
        out_shape=[
            jax.ShapeDtypeStruct((_N, 64), jnp.float32),
            jax.ShapeDtypeStruct((_N, 64), jnp.float32),
        ],
    )(p2, pc, h1, wl2t, bl2, wr2t, wl3t, wr3t)


def _tc_d(p3, pc, r3, bl3, w4t, b4):
    def body(p_ref, pc_ref, r_ref, bl_ref, w4_ref, b4_ref, o_ref):
        m3 = p_ref[0] + p_ref[1]
        inv = _inv_from_counts(pc_ref)
        h3 = jnp.maximum(m3 * inv + bl_ref[...] + r_ref[...], 0.0)
        o_ref[...] = jnp.dot(h3, w4_ref[...],
                             preferred_element_type=jnp.float32) + b4_ref[...]

    return pl.pallas_call(
        body,
        grid=(_N // _BM,),
        in_specs=[
            pl.BlockSpec((_NC, _BM, 64), lambda i: (0, i, 0)),
            pl.BlockSpec((_NC, _BM, _CW), lambda i: (0, i, 0)),
            pl.BlockSpec((_BM, 64), lambda i: (i, 0)),
            pl.BlockSpec((1, 64), lambda i: (0, 0)),
            pl.BlockSpec((64, 128), lambda i: (0, 0)),
            pl.BlockSpec((1, 128), lambda i: (0, 0)),
        ],
        out_specs=pl.BlockSpec((_BM, 128), lambda i: (i, 0)),
        out_shape=jax.ShapeDtypeStruct((_N, 128), jnp.float32),
    )(p3, pc, r3, bl3, w4t, b4)


def kernel(x, edge_index, Wl1, bl1, Wr1, Wl2, bl2, Wr2, Wl3, bl3, Wr3, W4, b4):
    e3 = edge_index.reshape(2, _NW, _EPW)
    ar = jnp.arange(_NDUM, dtype=jnp.int32)
    dsrc = jnp.broadcast_to((ar * 89) % _N, (_NW, _NDUM))
    ddst = jnp.broadcast_to(_N + (ar % _NSAC), (_NW, _NDUM))
    dummy = jnp.stack([dsrc, ddst])
    slab = jnp.concatenate([e3, dummy], axis=2)
    slab = slab.reshape(2, _NW, _NCH, _CK)
    slab = jnp.concatenate(
        [slab, jnp.zeros((2, _NW, _NCHP - _NCH, _CK), jnp.int32)], axis=2)
    epad = slab.reshape(2, _NW * _NCHP, _CK)
    src = epad[0]
    dst = epad[1]
    zeros64 = jnp.zeros((_N, 64), jnp.float32)
    zerosc = jnp.zeros((_N, _CW), jnp.float32)
    w4t = jnp.zeros((64, 128), jnp.float32).at[:, :3].set(W4.T)
    b4p = jnp.zeros((1, 128), jnp.float32).at[0, :3].set(b4)

    pc = _degree(dst, zerosc)
    y1, r1 = _tc_a(x, Wl1.T, Wr1.T)
    zeros64d, _ = lax.optimization_barrier((zeros64, pc))
    p1 = _segsum(y1, src, dst, zeros64d)
    h1 = _tc_b(p1, pc, r1, bl1.reshape(1, 64))
    p2 = _segsum(h1, src, dst, zeros64)
    y3, r3 = _tc_c(p2, pc, h1, Wl2.T, bl2.reshape(1, 128), Wr2.T, Wl3.T,
                   Wr3.T)
    p3 = _segsum(y3, src, dst, zeros64)
    out = _tc_d(p3, pc, r3, bl3.reshape(1, 64), w4t, b4p)
    return out[:, :3]


# speedup vs baseline: 17.3546x; 1.0770x over previous
import functools

import jax
import jax.numpy as jnp
from jax import lax
from jax.experimental import pallas as pl
from jax.experimental.pallas import tpu as pltpu
from jax.experimental.pallas import tpu_sc as plsc

_N = 10000
_E = 320000
_NC = 2
_NS = 16
_NW = _NC * _NS
_CK = 128
_EPW = _E // _NW
_NCH = 79
_NDUM = _NCH * _CK - _EPW
_NCHP = 80
_NSAC = 16
_RPT0 = 632
_RPT1 = _N - 15 * _RPT0
_CW = 32
_BM = 2000


_mesh = plsc.VectorSubcoreMesh(core_axis_name="c", subcore_axis_name="s")


@functools.partial(
    pl.kernel,
    mesh=_mesh,
    compiler_params=pltpu.CompilerParams(use_tc_tiling_on_sc=False),
    out_type=jax.ShapeDtypeStruct((_NC, _N, 64), jnp.float32),
    scratch_types=[
        pltpu.VMEM((_NCHP, _CK), jnp.int32),
        pltpu.VMEM((_NCHP, _CK), jnp.int32),
        pltpu.VMEM((_CK, 64), jnp.float32),
        pltpu.VMEM((_CK, 64), jnp.float32),
        pltpu.VMEM((_CK, 64), jnp.float32),
        pltpu.VMEM((_CK, 64), jnp.float32),
        pltpu.VMEM_SHARED((_N + _NSAC, 64), jnp.float32),
        pltpu.SemaphoreType.DMA,
        pltpu.SemaphoreType.DMA,
        pltpu.SemaphoreType.DMA,
        pltpu.SemaphoreType.DMA,
        pltpu.SemaphoreType.DMA,
        pltpu.SemaphoreType.DMA,
        pltpu.SemaphoreType.DMA,
        pltpu.SemaphoreType.DMA,
    ],
)
def _segsum(table, src2d, dst2d, zeros, out, sidx, didx, rows0, rows1, rows2,
            rows3, acc, gs0, gs1, gs2, gs3, ss0, ss1, ss2, ss3):
    c = lax.axis_index("c")
    s = lax.axis_index("s")
    wid = c * _NS + s
    row0 = s * _RPT0

    @pl.when(s < 15)
    def _():
        pltpu.sync_copy(zeros.at[pl.ds(row0, _RPT0)],
                        acc.at[pl.ds(row0, _RPT0)])

    @pl.when(s == 15)
    def _():
        pltpu.sync_copy(zeros.at[pl.ds(row0, _RPT1)],
                        acc.at[pl.ds(row0, _RPT1)])

    pltpu.sync_copy(src2d.at[pl.ds(wid * _NCHP, _NCHP)], sidx)
    pltpu.sync_copy(dst2d.at[pl.ds(wid * _NCHP, _NCHP)], didx)
    plsc.subcore_barrier()

    bufs = (rows0, rows1, rows2, rows3)
    gsems = (gs0, gs1, gs2, gs3)
    ssems = (ss0, ss1, ss2, ss3)

    pltpu.async_copy(table.at[sidx.at[0]], bufs[0], gsems[0])
    pltpu.async_copy(table.at[sidx.at[1]], bufs[1], gsems[1])

    def body(i, carry):
        j = i + 2
        for b in range(4):
            @pl.when((i % 4) == b)
            def _(b=b):
                pltpu.make_async_copy(table.at[sidx.at[i]], bufs[b],
                                      gsems[b]).wait()
                pltpu.async_copy(bufs[b], acc.at[didx.at[i]], ssems[b],
                                 add=True)
        for b in range(4):
            cond = jnp.logical_and(j < _NCH, (j % 4) == b)

            @pl.when(jnp.logical_and(cond, i >= 2))
            def _(b=b):
                pltpu.make_async_copy(bufs[b], acc.at[didx.at[0]],
                                      ssems[b]).wait()
                pltpu.async_copy(table.at[sidx.at[j]], bufs[b], gsems[b])

            @pl.when(jnp.logical_and(cond, i < 2))
            def _(b=b):
                pltpu.async_copy(table.at[sidx.at[j]], bufs[b], gsems[b])
        return carry

    lax.fori_loop(0, _NCH, body, 0)
    for b in range(4):
        pltpu.make_async_copy(bufs[b], acc.at[didx.at[0]], ssems[b]).wait()
    plsc.subcore_barrier()

    @pl.when(s < 15)
    def _():
        pltpu.sync_copy(acc.at[pl.ds(row0, _RPT0)],
                        out.at[c, pl.ds(row0, _RPT0)])

    @pl.when(s == 15)
    def _():
        pltpu.sync_copy(acc.at[pl.ds(row0, _RPT1)],
                        out.at[c, pl.ds(row0, _RPT1)])


@functools.partial(
    pl.kernel,
    mesh=_mesh,
    compiler_params=pltpu.CompilerParams(use_tc_tiling_on_sc=False),
    out_type=jax.ShapeDtypeStruct((_NC, _N, _CW), jnp.float32),
    scratch_types=[
        pltpu.VMEM((_NCHP, _CK), jnp.int32),
        pltpu.VMEM((_CK, _CW), jnp.float32),
        pltpu.VMEM_SHARED((_N + _NSAC, _CW), jnp.float32),
        pltpu.SemaphoreType.DMA,
        pltpu.SemaphoreType.DMA,
        pltpu.SemaphoreType.DMA,
        pltpu.SemaphoreType.DMA,
    ],
)
def _degree(dst2d, zeros, out, didx, ones, acc, s0, s1, s2, s3):
    c = lax.axis_index("c")
    s = lax.axis_index("s")
    wid = c * _NS + s
    row0 = s * _RPT0

    @pl.when(s < 15)
    def _():
        pltpu.sync_copy(zeros.at[pl.ds(row0, _RPT0)],
                        acc.at[pl.ds(row0, _RPT0)])

    @pl.when(s == 15)
    def _():
        pltpu.sync_copy(zeros.at[pl.ds(row0, _RPT1)],
                        acc.at[pl.ds(row0, _RPT1)])

    pltpu.sync_copy(dst2d.at[pl.ds(wid * _NCHP, _NCHP)], didx)

    def fill(i, carry):
        for j in range(_CW // 16):
            ones[i, pl.ds(j * 16, 16)] = jnp.full((16,), 1.0, jnp.float32)
        return carry

    lax.fori_loop(0, _CK, fill, 0)
    plsc.subcore_barrier()

    sems = (s0, s1, s2, s3)

    def body(i, carry):
        for b in range(4):
            @pl.when(jnp.logical_and((i % 4) == b, i >= 4))
            def _(b=b):
                pltpu.make_async_copy(ones, acc.at[didx.at[0]],
                                      sems[b]).wait()
            @pl.when((i % 4) == b)
            def _(b=b):
                pltpu.async_copy(ones, acc.at[didx.at[i]], sems[b], add=True)
        return carry

    lax.fori_loop(0, _NCH, body, 0)
    for b in range(4):
        pltpu.make_async_copy(ones, acc.at[didx.at[0]], sems[b]).wait()
    plsc.subcore_barrier()

    @pl.when(s < 15)
    def _():
        pltpu.sync_copy(acc.at[pl.ds(row0, _RPT0)],
                        out.at[c, pl.ds(row0, _RPT0)])

    @pl.when(s == 15)
    def _():
        pltpu.sync_copy(acc.at[pl.ds(row0, _RPT1)],
                        out.at[c, pl.ds(row0, _RPT1)])


_B2 = 1000


def _tc_a(xe, xo, wl1t, wr1t):
    def body(xe_ref, xo_ref, wl_ref, wr_ref, y_ref, r_ref):
        xe_b = xe_ref[...]
        xo_b = xo_ref[...]
        wl = wl_ref[...]
        wr = wr_ref[...]
        y_ref[...] = jnp.concatenate(
            [jnp.dot(xe_b, wl, preferred_element_type=jnp.float32),
             jnp.dot(xo_b, wl, preferred_element_type=jnp.float32)], axis=1)
        r_ref[...] = jnp.concatenate(
            [jnp.dot(xe_b, wr, preferred_element_type=jnp.float32),
             jnp.dot(xo_b, wr, preferred_element_type=jnp.float32)], axis=1)

    return pl.pallas_call(
        body,
        grid=(_N // 2 // _B2,),
        in_specs=[
            pl.BlockSpec((_B2, 128), lambda i: (i, 0)),
            pl.BlockSpec((_B2, 128), lambda i: (i, 0)),
            pl.BlockSpec((128, 64), lambda i: (0, 0)),
            pl.BlockSpec((128, 64), lambda i: (0, 0)),
        ],
        out_specs=[
            pl.BlockSpec((_B2, 128), lambda i: (i, 0)),
            pl.BlockSpec((_B2, 128), lambda i: (i, 0)),
        ],
        out_shape=[
            jax.ShapeDtypeStruct((_N // 2, 128), jnp.float32),
            jax.ShapeDtypeStruct((_N // 2, 128), jnp.float32),
        ],
    )(xe, xo, wl1t, wr1t)


def _invs(pce_ref, pco_ref):
    inve = 1.0 / jnp.maximum(pce_ref[0] + pce_ref[1], 1.0)
    invo = 1.0 / jnp.maximum(pco_ref[0] + pco_ref[1], 1.0)
    invp = jnp.concatenate([jnp.broadcast_to(inve, (_B2, 64)),
                            jnp.broadcast_to(invo, (_B2, 64))], axis=1)
    return inve, invo, invp


_PSPEC = pl.BlockSpec((_NC, _B2, 128), lambda i: (0, i, 0))
_CSPEC = pl.BlockSpec((_NC, _B2, 1), lambda i: (0, i, 0))
_VSPEC = pl.BlockSpec((_B2, 128), lambda i: (i, 0))
_V = jax.ShapeDtypeStruct((_N // 2, 128), jnp.float32)


def _tc_b(p1v, pce, pco, r1p, bl1c):
    def body(p_ref, pce_ref, pco_ref, r_ref, bl_ref, h_ref):
        p = p_ref[0] + p_ref[1]
        _, _, invp = _invs(pce_ref, pco_ref)
        h_ref[...] = jnp.maximum(p * invp + bl_ref[...] + r_ref[...], 0.0)

    return pl.pallas_call(
        body,
        grid=(_N // 2 // _B2,),
        in_specs=[_PSPEC, _CSPEC, _CSPEC, _VSPEC,
                  pl.BlockSpec((1, 128), lambda i: (0, 0))],
        out_specs=_VSPEC,
        out_shape=_V,
    )(p1v, pce, pco, r1p, bl1c)


def _tc_c(p2v, pce, pco, h1p, wl2e, wl2o, bl2, wr2e, wr2o, wl3t, wr3t):
    def body(p_ref, pce_ref, pco_ref, h1_ref, wl2e_ref, wl2o_ref, bl2_ref,
             wr2e_ref, wr2o_ref, wl3_ref, wr3_ref, y3_ref, r3_ref):
        p = p_ref[0] + p_ref[1]
        inve, invo, _ = _invs(pce_ref, pco_ref)
        h1 = h1_ref[...]
        bl2 = bl2_ref[...]
        dot = lambda a, b: jnp.dot(a, b, preferred_element_type=jnp.float32)
        h2e = jnp.maximum(
            dot(p, wl2e_ref[...]) * inve + bl2 + dot(h1, wr2e_ref[...]), 0.0)
        h2o = jnp.maximum(
            dot(p, wl2o_ref[...]) * invo + bl2 + dot(h1, wr2o_ref[...]), 0.0)
        wl3 = wl3_ref[...]
        wr3 = wr3_ref[...]
        y3_ref[...] = jnp.concatenate([dot(h2e, wl3), dot(h2o, wl3)], axis=1)
        r3_ref[...] = jnp.concatenate([dot(h2e, wr3), dot(h2o, wr3)], axis=1)

    wspec = pl.BlockSpec((128, 128), lambda i: (0, 0))
    w64 = pl.BlockSpec((128, 64), lambda i: (0, 0))
    return pl.pallas_call(
        body,
        grid=(_N // 2 // _B2,),
        in_specs=[_PSPEC, _CSPEC, _CSPEC, _VSPEC, wspec, wspec,
                  pl.BlockSpec((1, 128), lambda i: (0, 0)),
                  wspec, wspec, w64, w64],
        out_specs=[_VSPEC, _VSPEC],
        out_shape=[_V, _V],
    )(p2v, pce, pco, h1p, wl2e, wl2o, bl2, wr2e, wr2o, wl3t, wr3t)


def _tc_d(p3v, pce, pco, r3p, bl3c, w4s, b4s):
    def body(p_ref, pce_ref, pco_ref, r_ref, bl_ref, w4_ref, b4_ref, o_ref):
        p = p_ref[0] + p_ref[1]
        _, _, invp = _invs(pce_ref, pco_ref)
        h3 = jnp.maximum(p * invp + bl_ref[...] + r_ref[...], 0.0)
        o_ref[...] = jnp.dot(h3, w4_ref[...],
                             preferred_element_type=jnp.float32) + b4_ref[...]

    return pl.pallas_call(
        body,
        grid=(_N // 2 // _B2,),
        in_specs=[_PSPEC, _CSPEC, _CSPEC, _VSPEC,
                  pl.BlockSpec((1, 128), lambda i: (0, 0)),
                  pl.BlockSpec((128, 8), lambda i: (0, 0)),
                  pl.BlockSpec((1, 8), lambda i: (0, 0))],
        out_specs=pl.BlockSpec((_B2, 8), lambda i: (i, 0)),
        out_shape=jax.ShapeDtypeStruct((_N // 2, 8), jnp.float32),
    )(p3v, pce, pco, r3p, bl3c, w4s, b4s)


def kernel(x, edge_index, Wl1, bl1, Wr1, Wl2, bl2, Wr2, Wl3, bl3, Wr3, W4, b4):
    e3 = edge_index.reshape(2, _NW, _EPW)
    ar = jnp.arange(_NDUM, dtype=jnp.int32)
    dsrc = jnp.broadcast_to((ar * 89) % _N, (_NW, _NDUM))
    ddst = jnp.broadcast_to(_N + (ar % _NSAC), (_NW, _NDUM))
    dummy = jnp.stack([dsrc, ddst])
    slab = jnp.concatenate([e3, dummy], axis=2)
    slab = slab.reshape(2, _NW, _NCH, _CK)
    slab = jnp.concatenate(
        [slab, jnp.zeros((2, _NW, _NCHP - _NCH, _CK), jnp.int32)], axis=2)
    epad = slab.reshape(2, _NW * _NCHP, _CK)
    src = epad[0]
    dst = epad[1]
    zeros64 = jnp.zeros((_N, 64), jnp.float32)
    zerosc = jnp.zeros((_N, _CW), jnp.float32)

    xe = x[0::2]
    xo = x[1::2]
    z64 = jnp.zeros((64, 128), jnp.float32)
    wl2e = jnp.concatenate([Wl2.T, z64], axis=0)
    wl2o = jnp.concatenate([z64, Wl2.T], axis=0)
    wr2e = jnp.concatenate([Wr2.T, z64], axis=0)
    wr2o = jnp.concatenate([z64, Wr2.T], axis=0)
    bl1c = jnp.concatenate([bl1, bl1]).reshape(1, 128)
    bl3c = jnp.concatenate([bl3, bl3]).reshape(1, 128)
    w4s = (jnp.zeros((128, 8), jnp.float32)
           .at[0:64, 0:3].set(W4.T).at[64:128, 4:7].set(W4.T))
    b4s = (jnp.zeros((1, 8), jnp.float32)
           .at[0, 0:3].set(b4).at[0, 4:7].set(b4))

    pc = _degree(dst, zerosc)
    pce = pc[:, 0::2, :1]
    pco = pc[:, 1::2, :1]
    y1p, r1p = _tc_a(xe, xo, Wl1.T, Wr1.T)
    zeros64d, _ = lax.optimization_barrier((zeros64, pc))
    p1 = _segsum(jnp.reshape(y1p, (_N, 64)), src, dst, zeros64d)
    h1p = _tc_b(jnp.reshape(p1, (_NC, _N // 2, 128)), pce, pco, r1p, bl1c)
    p2 = _segsum(jnp.reshape(h1p, (_N, 64)), src, dst, zeros64)
    y3p, r3p = _tc_c(jnp.reshape(p2, (_NC, _N // 2, 128)), pce, pco, h1p,
                     wl2e, wl2o, bl2.reshape(1, 128), wr2e, wr2o,
                     Wl3.T, Wr3.T)
    p3 = _segsum(jnp.reshape(y3p, (_N, 64)), src, dst, zeros64)
    op = _tc_d(jnp.reshape(p3, (_NC, _N // 2, 128)), pce, pco, r3p, bl3c,
               w4s, b4s)
    return jnp.reshape(op, (_N, 4))[:, :3]


# speedup vs baseline: 18.5466x; 1.0687x over previous
import functools

import jax
import jax.numpy as jnp
from jax import lax
from jax.experimental import pallas as pl
from jax.experimental.pallas import tpu as pltpu
from jax.experimental.pallas import tpu_sc as plsc

_N = 10000
_E = 320000
_NC = 2
_NS = 16
_NW = _NC * _NS
_CK = 128
_EPW = _E // _NW
_NCH = 79
_NDUM = _NCH * _CK - _EPW
_NCHP = 80
_NSAC = 16
_RPT0 = 632
_RPT1 = _N - 15 * _RPT0
_CW = 32
_BM = 2000


_mesh = plsc.VectorSubcoreMesh(core_axis_name="c", subcore_axis_name="s")


@functools.partial(
    pl.kernel,
    mesh=_mesh,
    compiler_params=pltpu.CompilerParams(use_tc_tiling_on_sc=False),
    out_type=jax.ShapeDtypeStruct((_NC, _N, 64), jnp.float32),
    scratch_types=[
        pltpu.VMEM((_NCHP, _CK), jnp.int32),
        pltpu.VMEM((_NCHP, _CK), jnp.int32),
        pltpu.VMEM((_CK, 64), jnp.float32),
        pltpu.VMEM((_CK, 64), jnp.float32),
        pltpu.VMEM((_CK, 64), jnp.float32),
        pltpu.VMEM((_CK, 64), jnp.float32),
        pltpu.VMEM_SHARED((_N + _NSAC, 64), jnp.float32),
        pltpu.SemaphoreType.DMA,
        pltpu.SemaphoreType.DMA,
        pltpu.SemaphoreType.DMA,
        pltpu.SemaphoreType.DMA,
        pltpu.SemaphoreType.DMA,
        pltpu.SemaphoreType.DMA,
        pltpu.SemaphoreType.DMA,
        pltpu.SemaphoreType.DMA,
    ],
)
def _segsum(table, src2d, dst2d, zeros, out, sidx, didx, rows0, rows1, rows2,
            rows3, acc, gs0, gs1, gs2, gs3, ss0, ss1, ss2, ss3):
    c = lax.axis_index("c")
    s = lax.axis_index("s")
    wid = c * _NS + s
    row0 = s * _RPT0

    @pl.when(s < 15)
    def _():
        pltpu.sync_copy(zeros.at[pl.ds(row0, _RPT0)],
                        acc.at[pl.ds(row0, _RPT0)])

    @pl.when(s == 15)
    def _():
        pltpu.sync_copy(zeros.at[pl.ds(row0, _RPT1)],
                        acc.at[pl.ds(row0, _RPT1)])

    pltpu.sync_copy(src2d.at[pl.ds(wid * _NCHP, _NCHP)], sidx)
    pltpu.sync_copy(dst2d.at[pl.ds(wid * _NCHP, _NCHP)], didx)
    plsc.subcore_barrier()

    bufs = (rows0, rows1, rows2, rows3)
    gsems = (gs0, gs1, gs2, gs3)
    ssems = (ss0, ss1, ss2, ss3)

    pltpu.async_copy(table.at[sidx.at[0]], bufs[0], gsems[0])
    pltpu.async_copy(table.at[sidx.at[1]], bufs[1], gsems[1])

    def body(i, carry):
        j = i + 2
        for b in range(4):
            @pl.when((i % 4) == b)
            def _(b=b):
                pltpu.make_async_copy(table.at[sidx.at[i]], bufs[b],
                                      gsems[b]).wait()
                pltpu.async_copy(bufs[b], acc.at[didx.at[i]], ssems[b],
                                 add=True)
        for b in range(4):
            cond = jnp.logical_and(j < _NCH, (j % 4) == b)

            @pl.when(jnp.logical_and(cond, i >= 2))
            def _(b=b):
                pltpu.make_async_copy(bufs[b], acc.at[didx.at[0]],
                                      ssems[b]).wait()
                pltpu.async_copy(table.at[sidx.at[j]], bufs[b], gsems[b])

            @pl.when(jnp.logical_and(cond, i < 2))
            def _(b=b):
                pltpu.async_copy(table.at[sidx.at[j]], bufs[b], gsems[b])
        return carry

    lax.fori_loop(0, _NCH, body, 0)
    for b in range(4):
        pltpu.make_async_copy(bufs[b], acc.at[didx.at[0]], ssems[b]).wait()
    plsc.subcore_barrier()

    @pl.when(s < 15)
    def _():
        pltpu.sync_copy(acc.at[pl.ds(row0, _RPT0)],
                        out.at[c, pl.ds(row0, _RPT0)])

    @pl.when(s == 15)
    def _():
        pltpu.sync_copy(acc.at[pl.ds(row0, _RPT1)],
                        out.at[c, pl.ds(row0, _RPT1)])


@functools.partial(
    pl.kernel,
    mesh=_mesh,
    compiler_params=pltpu.CompilerParams(use_tc_tiling_on_sc=False),
    out_type=jax.ShapeDtypeStruct((_NC, _N, _CW), jnp.float32),
    scratch_types=[
        pltpu.VMEM((_NCHP, _CK), jnp.int32),
        pltpu.VMEM((_CK, _CW), jnp.float32),
        pltpu.VMEM_SHARED((_N + _NSAC, _CW), jnp.float32),
        pltpu.SemaphoreType.DMA,
        pltpu.SemaphoreType.DMA,
        pltpu.SemaphoreType.DMA,
        pltpu.SemaphoreType.DMA,
    ],
)
def _degree(dst2d, zeros, out, didx, ones, acc, s0, s1, s2, s3):
    c = lax.axis_index("c")
    s = lax.axis_index("s")
    wid = c * _NS + s
    row0 = s * _RPT0

    @pl.when(s < 15)
    def _():
        pltpu.sync_copy(zeros.at[pl.ds(row0, _RPT0)],
                        acc.at[pl.ds(row0, _RPT0)])

    @pl.when(s == 15)
    def _():
        pltpu.sync_copy(zeros.at[pl.ds(row0, _RPT1)],
                        acc.at[pl.ds(row0, _RPT1)])

    pltpu.sync_copy(dst2d.at[pl.ds(wid * _NCHP, _NCHP)], didx)

    def fill(i, carry):
        for j in range(_CW // 16):
            ones[i, pl.ds(j * 16, 16)] = jnp.full((16,), 1.0, jnp.float32)
        return carry

    lax.fori_loop(0, _CK, fill, 0)
    plsc.subcore_barrier()

    sems = (s0, s1, s2, s3)

    def body(i, carry):
        for b in range(4):
            @pl.when(jnp.logical_and((i % 4) == b, i >= 4))
            def _(b=b):
                pltpu.make_async_copy(ones, acc.at[didx.at[0]],
                                      sems[b]).wait()
            @pl.when((i % 4) == b)
            def _(b=b):
                pltpu.async_copy(ones, acc.at[didx.at[i]], sems[b], add=True)
        return carry

    lax.fori_loop(0, _NCH, body, 0)
    for b in range(4):
        pltpu.make_async_copy(ones, acc.at[didx.at[0]], sems[b]).wait()
    plsc.subcore_barrier()

    @pl.when(s < 15)
    def _():
        pltpu.sync_copy(acc.at[pl.ds(row0, _RPT0)],
                        out.at[c, pl.ds(row0, _RPT0)])

    @pl.when(s == 15)
    def _():
        pltpu.sync_copy(acc.at[pl.ds(row0, _RPT1)],
                        out.at[c, pl.ds(row0, _RPT1)])


_B2 = 1000


def _tc_a(xp, wl1t, wr1t):
    def body(xp_ref, wl_ref, wr_ref, y_ref, r_ref):
        xp_b = xp_ref[...]
        xe_b = xp_b[:, :128]
        xo_b = xp_b[:, 128:]
        wl = wl_ref[...]
        wr = wr_ref[...]
        y_ref[...] = jnp.concatenate(
            [jnp.dot(xe_b, wl, preferred_element_type=jnp.float32),
             jnp.dot(xo_b, wl, preferred_element_type=jnp.float32)], axis=1)
        r_ref[...] = jnp.concatenate(
            [jnp.dot(xe_b, wr, preferred_element_type=jnp.float32),
             jnp.dot(xo_b, wr, preferred_element_type=jnp.float32)], axis=1)

    return pl.pallas_call(
        body,
        grid=(_N // 2 // _B2,),
        in_specs=[
            pl.BlockSpec((_B2, 256), lambda i: (i, 0)),
            pl.BlockSpec((128, 64), lambda i: (0, 0)),
            pl.BlockSpec((128, 64), lambda i: (0, 0)),
        ],
        out_specs=[
            pl.BlockSpec((_B2, 128), lambda i: (i, 0)),
            pl.BlockSpec((_B2, 128), lambda i: (i, 0)),
        ],
        out_shape=[
            jax.ShapeDtypeStruct((_N // 2, 128), jnp.float32),
            jax.ShapeDtypeStruct((_N // 2, 128), jnp.float32),
        ],
    )(xp, wl1t, wr1t)


def _invs(pce_ref, pco_ref):
    inve = 1.0 / jnp.maximum(pce_ref[0] + pce_ref[1], 1.0)
    invo = 1.0 / jnp.maximum(pco_ref[0] + pco_ref[1], 1.0)
    invp = jnp.concatenate([jnp.broadcast_to(inve, (_B2, 64)),
                            jnp.broadcast_to(invo, (_B2, 64))], axis=1)
    return inve, invo, invp


_PSPEC = pl.BlockSpec((_NC, _B2, 128), lambda i: (0, i, 0))
_CSPEC = pl.BlockSpec((_NC, _B2, 1), lambda i: (0, i, 0))
_VSPEC = pl.BlockSpec((_B2, 128), lambda i: (i, 0))
_V = jax.ShapeDtypeStruct((_N // 2, 128), jnp.float32)


def _tc_b(p1v, pce, pco, r1p, bl1c):
    def body(p_ref, pce_ref, pco_ref, r_ref, bl_ref, h_ref):
        p = p_ref[0] + p_ref[1]
        _, _, invp = _invs(pce_ref, pco_ref)
        h_ref[...] = jnp.maximum(p * invp + bl_ref[...] + r_ref[...], 0.0)

    return pl.pallas_call(
        body,
        grid=(_N // 2 // _B2,),
        in_specs=[_PSPEC, _CSPEC, _CSPEC, _VSPEC,
                  pl.BlockSpec((1, 128), lambda i: (0, 0))],
        out_specs=_VSPEC,
        out_shape=_V,
    )(p1v, pce, pco, r1p, bl1c)


def _tc_c(p2v, pce, pco, h1p, wl2e, wl2o, bl2, wr2e, wr2o, wl3t, wr3t):
    def body(p_ref, pce_ref, pco_ref, h1_ref, wl2e_ref, wl2o_ref, bl2_ref,
             wr2e_ref, wr2o_ref, wl3_ref, wr3_ref, y3_ref, r3_ref):
        p = p_ref[0] + p_ref[1]
        inve, invo, _ = _invs(pce_ref, pco_ref)
        h1 = h1_ref[...]
        bl2 = bl2_ref[...]
        dot = lambda a, b: jnp.dot(a, b, preferred_element_type=jnp.float32)
        h2e = jnp.maximum(
            dot(p, wl2e_ref[...]) * inve + bl2 + dot(h1, wr2e_ref[...]), 0.0)
        h2o = jnp.maximum(
            dot(p, wl2o_ref[...]) * invo + bl2 + dot(h1, wr2o_ref[...]), 0.0)
        wl3 = wl3_ref[...]
        wr3 = wr3_ref[...]
        y3_ref[...] = jnp.concatenate([dot(h2e, wl3), dot(h2o, wl3)], axis=1)
        r3_ref[...] = jnp.concatenate([dot(h2e, wr3), dot(h2o, wr3)], axis=1)

    wspec = pl.BlockSpec((128, 128), lambda i: (0, 0))
    w64 = pl.BlockSpec((128, 64), lambda i: (0, 0))
    return pl.pallas_call(
        body,
        grid=(_N // 2 // _B2,),
        in_specs=[_PSPEC, _CSPEC, _CSPEC, _VSPEC, wspec, wspec,
                  pl.BlockSpec((1, 128), lambda i: (0, 0)),
                  wspec, wspec, w64, w64],
        out_specs=[_VSPEC, _VSPEC],
        out_shape=[_V, _V],
    )(p2v, pce, pco, h1p, wl2e, wl2o, bl2, wr2e, wr2o, wl3t, wr3t)


def _tc_d(p3v, pce, pco, r3p, bl3c, w4s, b4s):
    def body(p_ref, pce_ref, pco_ref, r_ref, bl_ref, w4_ref, b4_ref, o_ref):
        p = p_ref[0] + p_ref[1]
        _, _, invp = _invs(pce_ref, pco_ref)
        h3 = jnp.maximum(p * invp + bl_ref[...] + r_ref[...], 0.0)
        o_ref[...] = jnp.dot(h3, w4_ref[...],
                             preferred_element_type=jnp.float32) + b4_ref[...]

    return pl.pallas_call(
        body,
        grid=(_N // 2 // _B2,),
        in_specs=[_PSPEC, _CSPEC, _CSPEC, _VSPEC,
                  pl.BlockSpec((1, 128), lambda i: (0, 0)),
                  pl.BlockSpec((128, 8), lambda i: (0, 0)),
                  pl.BlockSpec((1, 8), lambda i: (0, 0))],
        out_specs=pl.BlockSpec((_B2, 8), lambda i: (i, 0)),
        out_shape=jax.ShapeDtypeStruct((_N // 2, 8), jnp.float32),
    )(p3v, pce, pco, r3p, bl3c, w4s, b4s)


def kernel(x, edge_index, Wl1, bl1, Wr1, Wl2, bl2, Wr2, Wl3, bl3, Wr3, W4, b4):
    e3 = edge_index.reshape(2, _NW, _EPW)
    ar = jnp.arange(_NDUM, dtype=jnp.int32)
    dsrc = jnp.broadcast_to((ar * 89) % _N, (_NW, _NDUM))
    ddst = jnp.broadcast_to(_N + (ar % _NSAC), (_NW, _NDUM))
    dummy = jnp.stack([dsrc, ddst])
    slab = jnp.concatenate([e3, dummy], axis=2)
    slab = slab.reshape(2, _NW, _NCH, _CK)
    slab = jnp.concatenate(
        [slab, jnp.zeros((2, _NW, _NCHP - _NCH, _CK), jnp.int32)], axis=2)
    epad = slab.reshape(2, _NW * _NCHP, _CK)
    src = epad[0]
    dst = epad[1]
    zeros64 = jnp.zeros((_N, 64), jnp.float32)
    zerosc = jnp.zeros((_N, _CW), jnp.float32)

    xp = jnp.reshape(x, (_N // 2, 256))
    z64 = jnp.zeros((64, 128), jnp.float32)
    wl2e = jnp.concatenate([Wl2.T, z64], axis=0)
    wl2o = jnp.concatenate([z64, Wl2.T], axis=0)
    wr2e = jnp.concatenate([Wr2.T, z64], axis=0)
    wr2o = jnp.concatenate([z64, Wr2.T], axis=0)
    bl1c = jnp.concatenate([bl1, bl1]).reshape(1, 128)
    bl3c = jnp.concatenate([bl3, bl3]).reshape(1, 128)
    w4s = (jnp.zeros((128, 8), jnp.float32)
           .at[0:64, 0:3].set(W4.T).at[64:128, 4:7].set(W4.T))
    b4s = (jnp.zeros((1, 8), jnp.float32)
           .at[0, 0:3].set(b4).at[0, 4:7].set(b4))

    pc = _degree(dst, zerosc)
    pce = pc[:, 0::2, :1]
    pco = pc[:, 1::2, :1]
    y1p, r1p = _tc_a(xp, Wl1.T, Wr1.T)
    zeros64d, _ = lax.optimization_barrier((zeros64, jnp.ravel(pc)))
    p1 = _segsum(jnp.reshape(y1p, (_N, 64)), src, dst, zeros64d)
    h1p = _tc_b(jnp.reshape(p1, (_NC, _N // 2, 128)), pce, pco, r1p, bl1c)
    p2 = _segsum(jnp.reshape(h1p, (_N, 64)), src, dst, zeros64)
    y3p, r3p = _tc_c(jnp.reshape(p2, (_NC, _N // 2, 128)), pce, pco, h1p,
                     wl2e, wl2o, bl2.reshape(1, 128), wr2e, wr2o,
                     Wl3.T, Wr3.T)
    p3 = _segsum(jnp.reshape(y3p, (_N, 64)), src, dst, zeros64)
    op = _tc_d(jnp.reshape(p3, (_NC, _N // 2, 128)), pce, pco, r3p, bl3c,
               w4s, b4s)
    return jnp.reshape(op, (_N, 4))[:, :3]
